# baseline (speedup 1.0000x reference)
import functools

import numpy as np

import jax
import jax.numpy as jnp
from jax import lax
from jax.experimental import pallas as pl
from jax.experimental.pallas import tpu as pltpu
from jax.experimental.pallas import tpu_sc as plsc

N = 10000
E = 160000
D = 256
DE = 16
H = 256
OUT = 128
DEPTH = 3
G = 64

NC = 2
NS = 16
EPT = E // NS
ROWS_PT = N // NS
K = 80
SK = 2000
CPS = SK // K
NSUP = EPT // SK
ER = E // K

BN = 2000
BE = 2000
F32 = jnp.float32
BF16 = jnp.bfloat16

_PERM = np.empty((2 * (H // 2),), np.int32)
for _p in range(2 * (H // 2)):
    _c, _r = divmod(_p, H // 2)
    _g, _q = divmod(_r, 32)
    _f = 2 * _q if _q < 16 else 2 * (_q - 16) + 1
    _PERM[_p] = 128 * _c + 32 * _g + _f


def _embed_body(x_ref, w1_ref, b1_ref, w2_ref, b2_ref, o_ref):
    h = jnp.maximum(
        jnp.dot(x_ref[...], w1_ref[...], preferred_element_type=F32) + b1_ref[0],
        0.0)
    o_ref[...] = jnp.dot(h, w2_ref[...], preferred_element_type=F32) + b2_ref[0]


def _embed(x, W1, b1, W2, b2):
    return pl.pallas_call(
        _embed_body,
        grid=(N // BN,),
        in_specs=[
            pl.BlockSpec((BN, D), lambda i: (i, 0)),
            pl.BlockSpec((D, H), lambda i: (0, 0)),
            pl.BlockSpec((1, H), lambda i: (0, 0)),
            pl.BlockSpec((H, H), lambda i: (0, 0)),
            pl.BlockSpec((1, H), lambda i: (0, 0)),
        ],
        out_specs=pl.BlockSpec((BN, H), lambda i: (i, 0)),
        out_shape=jax.ShapeDtypeStruct((N, H), F32),
    )(x, W1, b1, W2, b2)


def _q_body(ea_ref, wc_ref, q_ref):
    q_ref[...] = jnp.dot(ea_ref[...], wc_ref[0],
                         preferred_element_type=F32)[None, None].astype(BF16)


def _q_tables(edge_attr, Wm1c):
    return pl.pallas_call(
        _q_body,
        grid=(DEPTH, 2, E // BE),
        in_specs=[
            pl.BlockSpec((BE, DE), lambda i, c, e: (e, 0)),
            pl.BlockSpec((1, DE, H // 2), lambda i, c, e: (i, 0, c)),
        ],
        out_specs=pl.BlockSpec((1, 1, BE, H // 2), lambda i, c, e: (i, c, e, 0)),
        out_shape=jax.ShapeDtypeStruct((DEPTH, 2, E, H // 2), BF16),
    )(edge_attr, Wm1c)


def _pre_body(x_ref, wa_ref, wb_ref, bm_ref, pa_ref, pb_ref):
    xb = x_ref[...]
    pa_ref[...] = jnp.dot(
        xb, wa_ref[...], preferred_element_type=F32).astype(BF16)
    pb_ref[...] = (jnp.dot(xb, wb_ref[...], preferred_element_type=F32)
                   + bm_ref[0]).astype(BF16)


def _pre(x, Wa, Wb, bm):
    return pl.pallas_call(
        _pre_body,
        grid=(N // BN,),
        in_specs=[
            pl.BlockSpec((BN, H), lambda i: (i, 0)),
            pl.BlockSpec((H, H), lambda i: (0, 0)),
            pl.BlockSpec((H, H), lambda i: (0, 0)),
            pl.BlockSpec((1, H), lambda i: (0, 0)),
        ],
        out_specs=[
            pl.BlockSpec((BN, H), lambda i: (i, 0)),
            pl.BlockSpec((BN, H), lambda i: (i, 0)),
        ],
        out_shape=[
            jax.ShapeDtypeStruct((N, H), BF16),
            jax.ShapeDtypeStruct((N, H), BF16),
        ],
    )(x, Wa, Wb, bm)


def _upd_body(s_ref, cnt_ref, x_ref, wm2_ref, bm2_ref, wua_ref, wub_ref,
              bu1_ref, wu2_ref, bu2_ref, o_ref):
    s0 = s_ref[0]
    s1 = s_ref[1]
    ssum = (jnp.dot(s0, wm2_ref[0:128, :], preferred_element_type=F32)
            + jnp.dot(s1, wm2_ref[128:256, :], preferred_element_type=F32))
    cnt = cnt_ref[...][:, 0:1]
    aggr = (ssum + cnt * bm2_ref[0]) / jnp.maximum(cnt, 1.0)
    xb = x_ref[...]
    h = jnp.maximum(
        jnp.dot(xb, wua_ref[...], preferred_element_type=F32)
        + jnp.dot(aggr, wub_ref[...], preferred_element_type=F32)
        + bu1_ref[0], 0.0)
    o_ref[...] = jnp.dot(h, wu2_ref[...], preferred_element_type=F32) + bu2_ref[0]


def _update(S, cnt16, x, Wm2i, bm2i, Wua, Wub, bu1i, Wu2i, bu2i):
    return pl.pallas_call(
        _upd_body,
        grid=(N // BN,),
        in_specs=[
            pl.BlockSpec((2, BN, H // 2), lambda i: (0, i, 0)),
            pl.BlockSpec((BN, 16), lambda i: (i, 0)),
            pl.BlockSpec((BN, H), lambda i: (i, 0)),
            pl.BlockSpec((H, H), lambda i: (0, 0)),
            pl.BlockSpec((1, H), lambda i: (0, 0)),
            pl.BlockSpec((H, H), lambda i: (0, 0)),
            pl.BlockSpec((H, H), lambda i: (0, 0)),
            pl.BlockSpec((1, H), lambda i: (0, 0)),
            pl.BlockSpec((H, H), lambda i: (0, 0)),
            pl.BlockSpec((1, H), lambda i: (0, 0)),
        ],
        out_specs=pl.BlockSpec((BN, H), lambda i: (i, 0)),
        out_shape=jax.ShapeDtypeStruct((N, H), F32),
    )(S, cnt16, x, Wm2i, bm2i, Wua, Wub, bu1i, Wu2i, bu2i)


def _pool_body(x_ref, bid_ref, wh1_ref, bh1_ref, wh2_ref, bh2_ref, o_ref,
               acc_ref):
    i = pl.program_id(0)

    @pl.when(i == 0)
    def _init():
        acc_ref[...] = jnp.zeros_like(acc_ref)

    bid = bid_ref[0, 0]
    oh = (lax.broadcasted_iota(jnp.int32, (G, BN), 0)
          == bid[None, :]).astype(F32)
    acc_ref[...] += jnp.dot(oh, x_ref[...], preferred_element_type=F32)

    @pl.when(i == pl.num_programs(0) - 1)
    def _fin():
        h = jnp.maximum(
            jnp.dot(acc_ref[...], wh1_ref[...], preferred_element_type=F32)
            + bh1_ref[0], 0.0)
        o_ref[...] = jnp.dot(h, wh2_ref[...], preferred_element_type=F32) + bh2_ref[0]


def _pool_head(x, bidr, Wh1, bh1, Wh2, bh2):
    return pl.pallas_call(
        _pool_body,
        grid=(N // BN,),
        in_specs=[
            pl.BlockSpec((BN, H), lambda i: (i, 0)),
            pl.BlockSpec((1, 1, BN), lambda i: (i, 0, 0)),
            pl.BlockSpec((H, H), lambda i: (0, 0)),
            pl.BlockSpec((1, H), lambda i: (0, 0)),
            pl.BlockSpec((H, OUT), lambda i: (0, 0)),
            pl.BlockSpec((1, OUT), lambda i: (0, 0)),
        ],
        out_specs=pl.BlockSpec((G, OUT), lambda i: (0, 0)),
        out_shape=jax.ShapeDtypeStruct((G, OUT), F32),
        scratch_shapes=[pltpu.VMEM((G, H), F32)],
    )(x, bidr, Wh1, bh1, Wh2, bh2)


_MESH = plsc.VectorSubcoreMesh(core_axis_name="c", subcore_axis_name="s",
                               num_cores=NC, num_subcores=NS)
_SC_PARAMS = pltpu.CompilerParams(use_tc_tiling_on_sc=False,
                                  needs_layout_passes=False)


def _cnt_body(dstr_hbm, zc_hbm, cnt_out, sdst, vones, c_sh):
    cid = lax.axis_index("c")
    sid = lax.axis_index("s")
    myrows = pl.ds(sid * ROWS_PT, ROWS_PT)

    @pl.when(cid == 0)
    def _work():
        pltpu.sync_copy(zc_hbm.at[myrows], c_sh.at[myrows])

        def _ones_row(r, carry):
            vones[r] = jnp.ones((16,), F32)
            return carry
        lax.fori_loop(0, K, _ones_row, 0)
    plsc.subcore_barrier()

    @pl.when(cid == 0)
    def _scat():
        def chunk(c, carry):
            row = sid * (EPT // K) + c
            pltpu.sync_copy(dstr_hbm.at[pl.ds(row, 1)], sdst)
            pltpu.sync_copy(vones, c_sh.at[sdst.at[0]], add=True)
            return carry
        lax.fori_loop(0, EPT // K, chunk, 0)
    plsc.subcore_barrier()

    @pl.when(cid == 0)
    def _out():
        pltpu.sync_copy(c_sh.at[myrows], cnt_out.at[myrows])


_cnt_kernel = pl.kernel(
    _cnt_body,
    out_type=jax.ShapeDtypeStruct((N, 16), F32),
    mesh=_MESH,
    scratch_types=[
        pltpu.VMEM((1, K), jnp.int32),
        pltpu.VMEM((K, 16), F32),
        pltpu.VMEM_SHARED((N, 16), F32),
    ],
    compiler_params=_SC_PARAMS)


def _make_sc(layer):
    scratch = [
        pltpu.VMEM((CPS, K), jnp.int32),
        pltpu.VMEM((CPS, K), jnp.int32),
        pltpu.VMEM((CPS, K), jnp.int32),
        pltpu.VMEM((2, K, H // 2), BF16),
        pltpu.VMEM((2, K, H // 2), BF16),
        pltpu.VMEM((2, K, H // 2), BF16),
        pltpu.VMEM((K, H // 2), F32),
        pltpu.VMEM_SHARED((N, H // 2), F32),
        pltpu.SemaphoreType.DMA,
        pltpu.SemaphoreType.DMA,
    ]

    def body(pa_hbm, pb_hbm, qall_hbm, gsrc_hbm, gdst_hbm, dstr_hbm, z_hbm,
             s_out, isrc, idst, sdst, va, vb, vq, stg, s_sh, sem1, sem2):
        cid = lax.axis_index("c")
        sid = lax.axis_index("s")
        myrows = pl.ds(sid * ROWS_PT, ROWS_PT)

        pltpu.sync_copy(z_hbm.at[myrows], s_sh.at[myrows])
        plsc.subcore_barrier()

        def fire(sup_base_e, b):
            buf = b % 2
            cps = [
                pltpu.async_copy(pa_hbm.at[isrc.at[b]], va.at[buf], sem1),
                pltpu.async_copy(pb_hbm.at[idst.at[b]], vb.at[buf], sem1),
                pltpu.async_copy(
                    qall_hbm.at[layer, cid, pl.ds(sup_base_e + b * K, K)],
                    vq.at[buf], sem2),
            ]
            return cps

        def crunch(b):
            buf = b % 2

            def rowf(r, rc):
                for g in range(H // 2 // 32):
                    sl = pl.ds(g * 32, 32)
                    v = jnp.maximum(
                        va[buf, r, sl] + vb[buf, r, sl] + vq[buf, r, sl],
                        jnp.zeros((32,), BF16))
                    lo, hi = plsc.unpack(v, format=plsc.PackFormat.INTERLEAVED)
                    stg[r, pl.ds(g * 32, 16)] = lo
                    stg[r, pl.ds(g * 32 + 16, 16)] = hi
                return rc
            lax.fori_loop(0, K, rowf, 0)
            pltpu.sync_copy(stg, s_sh.at[sdst.at[b]], add=True)

        def super_loop(s, carry):
            base_row = sid * (EPT // K) + s * CPS
            base_e = sid * EPT + s * SK
            pltpu.sync_copy(gsrc_hbm.at[cid, pl.ds(base_row, CPS)], isrc)
            pltpu.sync_copy(gdst_hbm.at[cid, pl.ds(base_row, CPS)], idst)
            pltpu.sync_copy(dstr_hbm.at[pl.ds(base_row, CPS)], sdst)
            cps = fire(base_e, 0)
            for b in range(CPS):
                for cp in cps:
                    cp.wait()
                if b + 1 < CPS:
                    cps = fire(base_e, b + 1)
                crunch(b)
            return carry
        lax.fori_loop(0, NSUP, super_loop, 0)
        plsc.subcore_barrier()

        pltpu.sync_copy(s_sh.at[myrows], s_out.at[cid, myrows])

    return pl.kernel(body,
                     out_type=jax.ShapeDtypeStruct((2, N, H // 2), F32),
                     mesh=_MESH, scratch_types=scratch,
                     compiler_params=_SC_PARAMS)


_sc_layers = [_make_sc(i) for i in range(DEPTH)]


def kernel(x, edge_index, edge_attr, batch_ids, We1, be1, We2, be2,
           Wm1, bm1, Wm2, bm2, Wu1, bu1, Wu2, bu2, Wh1, bh1, Wh2, bh2):
    src = edge_index[0].astype(jnp.int32)
    dst = edge_index[1].astype(jnp.int32)
    gsrc = jnp.stack([2 * src, 2 * src + 1]).reshape(2, ER, K)
    gdst = jnp.stack([2 * dst, 2 * dst + 1]).reshape(2, ER, K)
    dstr = dst.reshape(ER, K)
    zrow = jnp.zeros((N, H // 2), F32)
    zc = jnp.zeros((N, 16), F32)
    bidr = batch_ids.astype(jnp.int32).reshape(N // BN, 1, BN)
    perm = jnp.asarray(_PERM)
    Wm2p = Wm2[:, perm, :]

    h = _embed(x, We1, be1.reshape(1, H), We2, be2.reshape(1, H))
    qall = _q_tables(edge_attr, Wm1[:, 2 * H:, :])
    cnt16 = _cnt_kernel(dstr, zc)

    for i in range(DEPTH):
        pa, pb = _pre(h, Wm1[i, :H, :], Wm1[i, H:2 * H, :],
                      bm1[i].reshape(1, H))
        pa2 = pa.reshape(2 * N, H // 2)
        pb2 = pb.reshape(2 * N, H // 2)
        S = _sc_layers[i](pa2, pb2, qall, gsrc, gdst, dstr, zrow)
        h = _update(S, cnt16, h, Wm2p[i], bm2[i].reshape(1, H),
                    Wu1[i, :H, :], Wu1[i, H:, :], bu1[i].reshape(1, H),
                    Wu2[i], bu2[i].reshape(1, H))

    return _pool_head(h, bidr, Wh1, bh1.reshape(1, H), Wh2, bh2.reshape(1, OUT))


# speedup vs baseline: 1.0014x; 1.0014x over previous
import functools

import numpy as np

import jax
import jax.numpy as jnp
from jax import lax
from jax.experimental import pallas as pl
from jax.experimental.pallas import tpu as pltpu
from jax.experimental.pallas import tpu_sc as plsc

N = 10000
E = 160000
D = 256
DE = 16
H = 256
OUT = 128
DEPTH = 3
G = 64

NC = 2
NS = 16
EPT = E // NS
ROWS_PT = N // NS
K = 80
SK = 2000
CPS = SK // K
NSUP = EPT // SK
ER = E // K

BN = 2000
BE = 2000
F32 = jnp.float32
BF16 = jnp.bfloat16


def _permute_wm2(Wm2):
    w = Wm2.reshape(DEPTH, H // 32, 16, 2, H)
    return jnp.transpose(w, (0, 1, 3, 2, 4)).reshape(DEPTH, H, H)


def _embed_body(x_ref, w1_ref, b1_ref, w2_ref, b2_ref, o_ref):
    h = jnp.maximum(
        jnp.dot(x_ref[...], w1_ref[...], preferred_element_type=F32) + b1_ref[0],
        0.0)
    o_ref[...] = jnp.dot(h, w2_ref[...], preferred_element_type=F32) + b2_ref[0]


def _embed(x, W1, b1, W2, b2):
    return pl.pallas_call(
        _embed_body,
        grid=(N // BN,),
        in_specs=[
            pl.BlockSpec((BN, D), lambda i: (i, 0)),
            pl.BlockSpec((D, H), lambda i: (0, 0)),
            pl.BlockSpec((1, H), lambda i: (0, 0)),
            pl.BlockSpec((H, H), lambda i: (0, 0)),
            pl.BlockSpec((1, H), lambda i: (0, 0)),
        ],
        out_specs=pl.BlockSpec((BN, H), lambda i: (i, 0)),
        out_shape=jax.ShapeDtypeStruct((N, H), F32),
    )(x, W1, b1, W2, b2)


def _q_body(ea_ref, wc_ref, q_ref):
    q_ref[...] = jnp.dot(ea_ref[...], wc_ref[0],
                         preferred_element_type=F32)[None, None].astype(BF16)


def _q_tables(edge_attr, Wm1c):
    return pl.pallas_call(
        _q_body,
        grid=(DEPTH, 2, E // BE),
        in_specs=[
            pl.BlockSpec((BE, DE), lambda i, c, e: (e, 0)),
            pl.BlockSpec((1, DE, H // 2), lambda i, c, e: (i, 0, c)),
        ],
        out_specs=pl.BlockSpec((1, 1, BE, H // 2), lambda i, c, e: (i, c, e, 0)),
        out_shape=jax.ShapeDtypeStruct((DEPTH, 2, E, H // 2), BF16),
    )(edge_attr, Wm1c)


def _pre_body(x_ref, wa_ref, wb_ref, bm_ref, pa_ref, pb_ref):
    xb = x_ref[...]
    for c in range(2):
        wc = pl.ds(c * (H // 2), H // 2)
        pa_ref[c] = jnp.dot(
            xb, wa_ref[:, wc], preferred_element_type=F32).astype(BF16)
        pb_ref[c] = (jnp.dot(xb, wb_ref[:, wc], preferred_element_type=F32)
                     + bm_ref[0, wc]).astype(BF16)


def _pre(x, Wa, Wb, bm):
    return pl.pallas_call(
        _pre_body,
        grid=(N // BN,),
        in_specs=[
            pl.BlockSpec((BN, H), lambda i: (i, 0)),
            pl.BlockSpec((H, H), lambda i: (0, 0)),
            pl.BlockSpec((H, H), lambda i: (0, 0)),
            pl.BlockSpec((1, H), lambda i: (0, 0)),
        ],
        out_specs=[
            pl.BlockSpec((2, BN, H // 2), lambda i: (0, i, 0)),
            pl.BlockSpec((2, BN, H // 2), lambda i: (0, i, 0)),
        ],
        out_shape=[
            jax.ShapeDtypeStruct((2, N, H // 2), BF16),
            jax.ShapeDtypeStruct((2, N, H // 2), BF16),
        ],
    )(x, Wa, Wb, bm)


def _upd_body(s_ref, cnt_ref, x_ref, wm2_ref, bm2_ref, wua_ref, wub_ref,
              bu1_ref, wu2_ref, bu2_ref, o_ref):
    s0 = s_ref[0]
    s1 = s_ref[1]
    ssum = (jnp.dot(s0, wm2_ref[0:128, :], preferred_element_type=F32)
            + jnp.dot(s1, wm2_ref[128:256, :], preferred_element_type=F32))
    cnt = (cnt_ref[0] + cnt_ref[1])[:, 0:1]
    aggr = (ssum + cnt * bm2_ref[0]) / jnp.maximum(cnt, 1.0)
    xb = x_ref[...]
    h = jnp.maximum(
        jnp.dot(xb, wua_ref[...], preferred_element_type=F32)
        + jnp.dot(aggr, wub_ref[...], preferred_element_type=F32)
        + bu1_ref[0], 0.0)
    o_ref[...] = jnp.dot(h, wu2_ref[...], preferred_element_type=F32) + bu2_ref[0]


def _update(S, cnt16, x, Wm2i, bm2i, Wua, Wub, bu1i, Wu2i, bu2i):
    return pl.pallas_call(
        _upd_body,
        grid=(N // BN,),
        in_specs=[
            pl.BlockSpec((2, BN, H // 2), lambda i: (0, i, 0)),
            pl.BlockSpec((NC, BN, 16), lambda i: (0, i, 0)),
            pl.BlockSpec((BN, H), lambda i: (i, 0)),
            pl.BlockSpec((H, H), lambda i: (0, 0)),
            pl.BlockSpec((1, H), lambda i: (0, 0)),
            pl.BlockSpec((H, H), lambda i: (0, 0)),
            pl.BlockSpec((H, H), lambda i: (0, 0)),
            pl.BlockSpec((1, H), lambda i: (0, 0)),
            pl.BlockSpec((H, H), lambda i: (0, 0)),
            pl.BlockSpec((1, H), lambda i: (0, 0)),
        ],
        out_specs=pl.BlockSpec((BN, H), lambda i: (i, 0)),
        out_shape=jax.ShapeDtypeStruct((N, H), F32),
    )(S, cnt16, x, Wm2i, bm2i, Wua, Wub, bu1i, Wu2i, bu2i)


def _pool_body(x_ref, bid_ref, wh1_ref, bh1_ref, wh2_ref, bh2_ref, o_ref,
               acc_ref):
    i = pl.program_id(0)

    @pl.when(i == 0)
    def _init():
        acc_ref[...] = jnp.zeros_like(acc_ref)

    bid = bid_ref[0, 0]
    oh = (lax.broadcasted_iota(jnp.int32, (G, BN), 0)
          == bid[None, :]).astype(F32)
    acc_ref[...] += jnp.dot(oh, x_ref[...], preferred_element_type=F32)

    @pl.when(i == pl.num_programs(0) - 1)
    def _fin():
        h = jnp.maximum(
            jnp.dot(acc_ref[...], wh1_ref[...], preferred_element_type=F32)
            + bh1_ref[0], 0.0)
        o_ref[...] = jnp.dot(h, wh2_ref[...], preferred_element_type=F32) + bh2_ref[0]


def _pool_head(x, bidr, Wh1, bh1, Wh2, bh2):
    return pl.pallas_call(
        _pool_body,
        grid=(N // BN,),
        in_specs=[
            pl.BlockSpec((BN, H), lambda i: (i, 0)),
            pl.BlockSpec((1, 1, BN), lambda i: (i, 0, 0)),
            pl.BlockSpec((H, H), lambda i: (0, 0)),
            pl.BlockSpec((1, H), lambda i: (0, 0)),
            pl.BlockSpec((H, OUT), lambda i: (0, 0)),
            pl.BlockSpec((1, OUT), lambda i: (0, 0)),
        ],
        out_specs=pl.BlockSpec((G, OUT), lambda i: (0, 0)),
        out_shape=jax.ShapeDtypeStruct((G, OUT), F32),
        scratch_shapes=[pltpu.VMEM((G, H), F32)],
    )(x, bidr, Wh1, bh1, Wh2, bh2)


_MESH = plsc.VectorSubcoreMesh(core_axis_name="c", subcore_axis_name="s",
                               num_cores=NC, num_subcores=NS)
_SC_PARAMS = pltpu.CompilerParams(use_tc_tiling_on_sc=False,
                                  needs_layout_passes=False)


_KC = 100
_CROWS = E // _KC // (NC * NS)


def _cnt_body(dstr2_hbm, zc_hbm, cnt_out, sdst, vones, c_sh):
    cid = lax.axis_index("c")
    sid = lax.axis_index("s")
    myrows = pl.ds(sid * ROWS_PT, ROWS_PT)

    pltpu.sync_copy(zc_hbm.at[myrows], c_sh.at[myrows])

    def _ones_row(r, carry):
        vones[r] = jnp.ones((16,), F32)
        return carry
    lax.fori_loop(0, _KC, _ones_row, 0)
    plsc.subcore_barrier()

    base = (cid * NS + sid) * _CROWS
    pltpu.sync_copy(dstr2_hbm.at[pl.ds(base, _CROWS)], sdst)

    def chunk(c, carry):
        pltpu.sync_copy(vones, c_sh.at[sdst.at[c]], add=True)
        return carry
    lax.fori_loop(0, _CROWS, chunk, 0)
    plsc.subcore_barrier()

    pltpu.sync_copy(c_sh.at[myrows], cnt_out.at[cid, myrows])


_cnt_kernel = pl.kernel(
    _cnt_body,
    out_type=jax.ShapeDtypeStruct((NC, N, 16), F32),
    mesh=_MESH,
    scratch_types=[
        pltpu.VMEM((_CROWS, _KC), jnp.int32),
        pltpu.VMEM((_KC, 16), F32),
        pltpu.VMEM_SHARED((N, 16), F32),
    ],
    compiler_params=_SC_PARAMS)


def _make_sc(layer):
    scratch = [
        pltpu.VMEM((CPS, K), jnp.int32),
        pltpu.VMEM((CPS, K), jnp.int32),
        pltpu.VMEM((CPS, K), jnp.int32),
        pltpu.VMEM((2, K, H // 2), BF16),
        pltpu.VMEM((2, K, H // 2), BF16),
        pltpu.VMEM((2, K, H // 2), BF16),
        pltpu.VMEM((K, H // 2), F32),
        pltpu.VMEM_SHARED((N, H // 2), F32),
        pltpu.SemaphoreType.DMA,
        pltpu.SemaphoreType.DMA,
    ]

    def body(pa_hbm, pb_hbm, qall_hbm, gsrc_hbm, gdst_hbm, dstr_hbm, z_hbm,
             s_out, isrc, idst, sdst, va, vb, vq, stg, s_sh, sem1, sem2):
        cid = lax.axis_index("c")
        sid = lax.axis_index("s")
        myrows = pl.ds(sid * ROWS_PT, ROWS_PT)

        pltpu.sync_copy(z_hbm.at[myrows], s_sh.at[myrows])
        plsc.subcore_barrier()

        def fire(sup_base_e, b):
            buf = b % 2
            cps = [
                pltpu.async_copy(pa_hbm.at[isrc.at[b]], va.at[buf], sem1),
                pltpu.async_copy(pb_hbm.at[idst.at[b]], vb.at[buf], sem1),
                pltpu.async_copy(
                    qall_hbm.at[layer, cid, pl.ds(sup_base_e + b * K, K)],
                    vq.at[buf], sem2),
            ]
            return cps

        def crunch(b):
            buf = b % 2

            def rowf(r, rc):
                for g in range(H // 2 // 32):
                    sl = pl.ds(g * 32, 32)
                    v = jnp.maximum(
                        va[buf, r, sl] + vb[buf, r, sl] + vq[buf, r, sl],
                        jnp.zeros((32,), BF16))
                    lo, hi = plsc.unpack(v, format=plsc.PackFormat.INTERLEAVED)
                    stg[r, pl.ds(g * 32, 16)] = lo
                    stg[r, pl.ds(g * 32 + 16, 16)] = hi
                return rc
            lax.fori_loop(0, K, rowf, 0)
            pltpu.sync_copy(stg, s_sh.at[sdst.at[b]], add=True)

        def super_loop(s, carry):
            base_row = sid * (EPT // K) + s * CPS
            base_e = sid * EPT + s * SK
            pltpu.sync_copy(gsrc_hbm.at[cid, pl.ds(base_row, CPS)], isrc)
            pltpu.sync_copy(gdst_hbm.at[cid, pl.ds(base_row, CPS)], idst)
            pltpu.sync_copy(dstr_hbm.at[pl.ds(base_row, CPS)], sdst)
            cps = fire(base_e, 0)
            for b in range(CPS):
                for cp in cps:
                    cp.wait()
                if b + 1 < CPS:
                    cps = fire(base_e, b + 1)
                crunch(b)
            return carry
        lax.fori_loop(0, NSUP, super_loop, 0)
        plsc.subcore_barrier()

        pltpu.sync_copy(s_sh.at[myrows], s_out.at[cid, myrows])

    return pl.kernel(body,
                     out_type=jax.ShapeDtypeStruct((2, N, H // 2), F32),
                     mesh=_MESH, scratch_types=scratch,
                     compiler_params=_SC_PARAMS)


_sc_layers = [_make_sc(i) for i in range(DEPTH)]


def kernel(x, edge_index, edge_attr, batch_ids, We1, be1, We2, be2,
           Wm1, bm1, Wm2, bm2, Wu1, bu1, Wu2, bu2, Wh1, bh1, Wh2, bh2):
    src = edge_index[0].astype(jnp.int32)
    dst = edge_index[1].astype(jnp.int32)
    gsrc = jnp.stack([src, N + src]).reshape(2, ER, K)
    gdst = jnp.stack([dst, N + dst]).reshape(2, ER, K)
    dstr = dst.reshape(ER, K)
    dstr2 = dst.reshape(E // _KC, _KC)
    zrow = jnp.zeros((N, H // 2), F32)
    zc = jnp.zeros((N, 16), F32)
    bidr = batch_ids.astype(jnp.int32).reshape(N // BN, 1, BN)
    Wm2p = _permute_wm2(Wm2)

    h = _embed(x, We1, be1.reshape(1, H), We2, be2.reshape(1, H))
    qall = _q_tables(edge_attr, Wm1[:, 2 * H:, :])
    cnt16 = _cnt_kernel(dstr2, zc)

    for i in range(DEPTH):
        pa, pb = _pre(h, Wm1[i, :H, :], Wm1[i, H:2 * H, :],
                      bm1[i].reshape(1, H))
        pa2 = pa.reshape(2 * N, H // 2)
        pb2 = pb.reshape(2 * N, H // 2)
        S = _sc_layers[i](pa2, pb2, qall, gsrc, gdst, dstr, zrow)
        h = _update(S, cnt16, h, Wm2p[i], bm2[i].reshape(1, H),
                    Wu1[i, :H, :], Wu1[i, H:, :], bu1[i].reshape(1, H),
                    Wu2[i], bu2[i].reshape(1, H))

    return _pool_head(h, bidr, Wh1, bh1.reshape(1, H), Wh2, bh2.reshape(1, OUT))


# speedup vs baseline: 1.4865x; 1.4844x over previous
import functools

import numpy as np

import jax
import jax.numpy as jnp
from jax import lax
from jax.experimental import pallas as pl
from jax.experimental.pallas import tpu as pltpu
from jax.experimental.pallas import tpu_sc as plsc

N = 10000
E = 160000
D = 256
DE = 16
H = 256
OUT = 128
DEPTH = 3
G = 64

NC = 2
NS = 16
EPT = E // NS
ROWS_PT = N // NS
K = 80
SK = 2000
CPS = SK // K
NSUP = EPT // SK
ER = E // K

BN = 2000
BE = 640
F32 = jnp.float32
BF16 = jnp.bfloat16


def _permute_wm2(Wm2):
    w = Wm2.reshape(DEPTH, 2, 2, 4, 16, H)
    return jnp.transpose(w, (0, 1, 3, 2, 4, 5)).reshape(DEPTH, H, H)


def _pack_bf16_pairs(y):
    lo = jax.lax.bitcast_convert_type(
        y[:, :H // 4].astype(BF16), jnp.int16).astype(jnp.int32) & 0xFFFF
    hi = jax.lax.bitcast_convert_type(
        y[:, H // 4:].astype(BF16), jnp.int16).astype(jnp.int32) << 16
    return jax.lax.bitcast_convert_type(lo | hi, F32)


def _embed_body(x_ref, w1_ref, b1_ref, w2_ref, b2_ref, o_ref):
    h = jnp.maximum(
        jnp.dot(x_ref[...], w1_ref[...], preferred_element_type=F32) + b1_ref[0],
        0.0)
    o_ref[...] = jnp.dot(h, w2_ref[...], preferred_element_type=F32) + b2_ref[0]


def _embed(x, W1, b1, W2, b2):
    return pl.pallas_call(
        _embed_body,
        grid=(N // BN,),
        in_specs=[
            pl.BlockSpec((BN, D), lambda i: (i, 0)),
            pl.BlockSpec((D, H), lambda i: (0, 0)),
            pl.BlockSpec((1, H), lambda i: (0, 0)),
            pl.BlockSpec((H, H), lambda i: (0, 0)),
            pl.BlockSpec((1, H), lambda i: (0, 0)),
        ],
        out_specs=pl.BlockSpec((BN, H), lambda i: (i, 0)),
        out_shape=jax.ShapeDtypeStruct((N, H), F32),
    )(x, W1, b1, W2, b2)


def _q_body(eae_ref, eao_ref, wc_ref, q_ref):
    qe = jax.lax.dot_general(eae_ref[...], wc_ref[...],
                             (((0,), (0,)), ((), ())),
                             preferred_element_type=F32)
    qo = jax.lax.dot_general(eao_ref[...], wc_ref[...],
                             (((0,), (0,)), ((), ())),
                             preferred_element_type=F32)
    q_ref[...] = jnp.concatenate(
        [_pack_bf16_pairs(qe), _pack_bf16_pairs(qo)], axis=1)[None]


def _q_layer(eaE, eaO, wc):
    return pl.pallas_call(
        _q_body,
        grid=(2, E // 2 // BE),
        in_specs=[
            pl.BlockSpec((DE, BE), lambda c, e: (0, e)),
            pl.BlockSpec((DE, BE), lambda c, e: (0, e)),
            pl.BlockSpec((DE, H // 2), lambda c, e: (0, c)),
        ],
        out_specs=pl.BlockSpec((1, BE, H // 2), lambda c, e: (c, e, 0)),
        out_shape=jax.ShapeDtypeStruct((2, E // 2, H // 2), F32),
    )(eaE, eaO, wc)


def _pre_body(x_ref, wa_ref, wb_ref, bm_ref, pa_ref, pb_ref):
    xb = x_ref[...]
    for c in range(2):
        wc = pl.ds(c * (H // 2), H // 2)
        pa_ref[c] = _pack_bf16_pairs(
            jnp.dot(xb, wa_ref[:, wc], preferred_element_type=F32))
        pb_ref[c] = _pack_bf16_pairs(
            jnp.dot(xb, wb_ref[:, wc], preferred_element_type=F32)
            + bm_ref[0, wc])


def _pre(x, Wa, Wb, bm):
    return pl.pallas_call(
        _pre_body,
        grid=(N // BN,),
        in_specs=[
            pl.BlockSpec((BN, H), lambda i: (i, 0)),
            pl.BlockSpec((H, H), lambda i: (0, 0)),
            pl.BlockSpec((H, H), lambda i: (0, 0)),
            pl.BlockSpec((1, H), lambda i: (0, 0)),
        ],
        out_specs=[
            pl.BlockSpec((2, BN, H // 4), lambda i: (0, i, 0)),
            pl.BlockSpec((2, BN, H // 4), lambda i: (0, i, 0)),
        ],
        out_shape=[
            jax.ShapeDtypeStruct((2, N, H // 4), F32),
            jax.ShapeDtypeStruct((2, N, H // 4), F32),
        ],
    )(x, Wa, Wb, bm)


def _upd_body(s_ref, cnt_ref, x_ref, wm2_ref, bm2_ref, wua_ref, wub_ref,
              bu1_ref, wu2_ref, bu2_ref, o_ref):
    s0 = s_ref[0]
    s1 = s_ref[1]
    ssum = (jnp.dot(s0, wm2_ref[0:128, :], preferred_element_type=F32)
            + jnp.dot(s1, wm2_ref[128:256, :], preferred_element_type=F32))
    cnt = (cnt_ref[0] + cnt_ref[1])[:, 0:1]
    aggr = (ssum + cnt * bm2_ref[0]) / jnp.maximum(cnt, 1.0)
    xb = x_ref[...]
    h = jnp.maximum(
        jnp.dot(xb, wua_ref[...], preferred_element_type=F32)
        + jnp.dot(aggr, wub_ref[...], preferred_element_type=F32)
        + bu1_ref[0], 0.0)
    o_ref[...] = jnp.dot(h, wu2_ref[...], preferred_element_type=F32) + bu2_ref[0]


def _update(S, cnt16, x, Wm2i, bm2i, Wua, Wub, bu1i, Wu2i, bu2i):
    return pl.pallas_call(
        _upd_body,
        grid=(N // BN,),
        in_specs=[
            pl.BlockSpec((2, BN, H // 2), lambda i: (0, i, 0)),
            pl.BlockSpec((NC, BN, 16), lambda i: (0, i, 0)),
            pl.BlockSpec((BN, H), lambda i: (i, 0)),
            pl.BlockSpec((H, H), lambda i: (0, 0)),
            pl.BlockSpec((1, H), lambda i: (0, 0)),
            pl.BlockSpec((H, H), lambda i: (0, 0)),
            pl.BlockSpec((H, H), lambda i: (0, 0)),
            pl.BlockSpec((1, H), lambda i: (0, 0)),
            pl.BlockSpec((H, H), lambda i: (0, 0)),
            pl.BlockSpec((1, H), lambda i: (0, 0)),
        ],
        out_specs=pl.BlockSpec((BN, H), lambda i: (i, 0)),
        out_shape=jax.ShapeDtypeStruct((N, H), F32),
    )(S, cnt16, x, Wm2i, bm2i, Wua, Wub, bu1i, Wu2i, bu2i)


def _pool_body(x_ref, bid_ref, wh1_ref, bh1_ref, wh2_ref, bh2_ref, o_ref,
               acc_ref):
    i = pl.program_id(0)

    @pl.when(i == 0)
    def _init():
        acc_ref[...] = jnp.zeros_like(acc_ref)

    bid = bid_ref[0, 0]
    oh = (lax.broadcasted_iota(jnp.int32, (G, BN), 0)
          == bid[None, :]).astype(F32)
    acc_ref[...] += jnp.dot(oh, x_ref[...], preferred_element_type=F32)

    @pl.when(i == pl.num_programs(0) - 1)
    def _fin():
        h = jnp.maximum(
            jnp.dot(acc_ref[...], wh1_ref[...], preferred_element_type=F32)
            + bh1_ref[0], 0.0)
        o_ref[...] = jnp.dot(h, wh2_ref[...], preferred_element_type=F32) + bh2_ref[0]


def _pool_head(x, bidr, Wh1, bh1, Wh2, bh2):
    return pl.pallas_call(
        _pool_body,
        grid=(N // BN,),
        in_specs=[
            pl.BlockSpec((BN, H), lambda i: (i, 0)),
            pl.BlockSpec((1, 1, BN), lambda i: (i, 0, 0)),
            pl.BlockSpec((H, H), lambda i: (0, 0)),
            pl.BlockSpec((1, H), lambda i: (0, 0)),
            pl.BlockSpec((H, OUT), lambda i: (0, 0)),
            pl.BlockSpec((1, OUT), lambda i: (0, 0)),
        ],
        out_specs=pl.BlockSpec((G, OUT), lambda i: (0, 0)),
        out_shape=jax.ShapeDtypeStruct((G, OUT), F32),
        scratch_shapes=[pltpu.VMEM((G, H), F32)],
    )(x, bidr, Wh1, bh1, Wh2, bh2)


_MESH = plsc.VectorSubcoreMesh(core_axis_name="c", subcore_axis_name="s",
                               num_cores=NC, num_subcores=NS)
_SC_PARAMS = pltpu.CompilerParams(use_tc_tiling_on_sc=False,
                                  needs_layout_passes=False)


_KC = 100
_CROWS = E // _KC // (NC * NS)


def _cnt_body(dstr2_hbm, zc_hbm, cnt_out, sdst, vones, c_sh):
    cid = lax.axis_index("c")
    sid = lax.axis_index("s")
    myrows = pl.ds(sid * ROWS_PT, ROWS_PT)

    pltpu.sync_copy(zc_hbm.at[myrows], c_sh.at[myrows])

    def _ones_row(r, carry):
        vones[r] = jnp.ones((16,), F32)
        return carry
    lax.fori_loop(0, _KC, _ones_row, 0)
    plsc.subcore_barrier()

    base = (cid * NS + sid) * _CROWS
    pltpu.sync_copy(dstr2_hbm.at[pl.ds(base, _CROWS)], sdst)

    def chunk(c, carry):
        pltpu.sync_copy(vones, c_sh.at[sdst.at[c]], add=True)
        return carry
    lax.fori_loop(0, _CROWS, chunk, 0)
    plsc.subcore_barrier()

    pltpu.sync_copy(c_sh.at[myrows], cnt_out.at[cid, myrows])


_cnt_kernel = pl.kernel(
    _cnt_body,
    out_type=jax.ShapeDtypeStruct((NC, N, 16), F32),
    mesh=_MESH,
    scratch_types=[
        pltpu.VMEM((_CROWS, _KC), jnp.int32),
        pltpu.VMEM((_KC, 16), F32),
        pltpu.VMEM_SHARED((N, 16), F32),
    ],
    compiler_params=_SC_PARAMS)


def _make_sc(layer):
    scratch = [
        pltpu.VMEM((CPS, K), jnp.int32),
        pltpu.VMEM((CPS, K), jnp.int32),
        pltpu.VMEM((CPS, K), jnp.int32),
        pltpu.VMEM((2, K, H // 4), F32),
        pltpu.VMEM((2, K, H // 4), F32),
        pltpu.VMEM((2, K // 2, H // 2), F32),
        pltpu.VMEM((K, H // 2), F32),
        pltpu.VMEM_SHARED((N, H // 2), F32),
        pltpu.SemaphoreType.DMA,
        pltpu.SemaphoreType.DMA,
    ]

    def body(pa_hbm, pb_hbm, ql_hbm, gsrc_hbm, gdst_hbm, dstr_hbm, z_hbm,
             s_out, isrc, idst, sdst, va, vb, vq, stg, s_sh, sem1, sem2):
        cid = lax.axis_index("c")
        sid = lax.axis_index("s")
        myrows = pl.ds(sid * ROWS_PT, ROWS_PT)

        pltpu.sync_copy(z_hbm.at[myrows], s_sh.at[myrows])
        plsc.subcore_barrier()

        def fire(sup_base_e, b):
            buf = b % 2
            cps = [
                pltpu.async_copy(pa_hbm.at[isrc.at[b]], va.at[buf], sem1),
                pltpu.async_copy(pb_hbm.at[idst.at[b]], vb.at[buf], sem1),
                pltpu.async_copy(
                    ql_hbm.at[cid, pl.ds((sup_base_e + b * K) // 2, K // 2)],
                    vq.at[buf], sem2),
            ]
            return cps

        def crunch(b):
            buf = b % 2

            def rowf(r2, rc):
                for p in range(2):
                    r = 2 * r2 + p
                    for g in range(H // 2 // 32):
                        sl = pl.ds(g * 16, 16)
                        a32 = plsc.bitcast(va[buf, r, sl], BF16)
                        b32 = plsc.bitcast(vb[buf, r, sl], BF16)
                        q32 = plsc.bitcast(
                            vq[buf, r2, pl.ds(p * 64 + g * 16, 16)], BF16)
                        v = jnp.maximum(a32 + b32 + q32,
                                        jnp.zeros((32,), BF16))
                        lo, hi = plsc.unpack(
                            v, format=plsc.PackFormat.INTERLEAVED)
                        stg[r, pl.ds(g * 32, 16)] = lo
                        stg[r, pl.ds(g * 32 + 16, 16)] = hi
                return rc
            lax.fori_loop(0, K // 2, rowf, 0)
            pltpu.sync_copy(stg, s_sh.at[sdst.at[b]], add=True)

        def super_loop(s, carry):
            base_row = sid * (EPT // K) + s * CPS
            base_e = sid * EPT + s * SK
            pltpu.sync_copy(gsrc_hbm.at[cid, pl.ds(base_row, CPS)], isrc)
            pltpu.sync_copy(gdst_hbm.at[cid, pl.ds(base_row, CPS)], idst)
            pltpu.sync_copy(dstr_hbm.at[pl.ds(base_row, CPS)], sdst)
            cps = fire(base_e, 0)
            for b in range(CPS):
                for cp in cps:
                    cp.wait()
                if b + 1 < CPS:
                    cps = fire(base_e, b + 1)
                crunch(b)
            return carry
        lax.fori_loop(0, NSUP, super_loop, 0)
        plsc.subcore_barrier()

        pltpu.sync_copy(s_sh.at[myrows], s_out.at[cid, myrows])

    return pl.kernel(body,
                     out_type=jax.ShapeDtypeStruct((2, N, H // 2), F32),
                     mesh=_MESH, scratch_types=scratch,
                     compiler_params=_SC_PARAMS)


_sc_layers = [_make_sc(i) for i in range(DEPTH)]


def kernel(x, edge_index, edge_attr, batch_ids, We1, be1, We2, be2,
           Wm1, bm1, Wm2, bm2, Wu1, bu1, Wu2, bu2, Wh1, bh1, Wh2, bh2):
    src = edge_index[0].astype(jnp.int32)
    dst = edge_index[1].astype(jnp.int32)
    gsrc = jnp.stack([src, N + src]).reshape(2, ER, K)
    gdst = jnp.stack([dst, N + dst]).reshape(2, ER, K)
    dstr = dst.reshape(ER, K)
    dstr2 = dst.reshape(E // _KC, _KC)
    zrow = jnp.zeros((N, H // 2), F32)
    zc = jnp.zeros((N, 16), F32)
    bidr = batch_ids.astype(jnp.int32).reshape(N // BN, 1, BN)
    Wm2p = _permute_wm2(Wm2)

    h = _embed(x, We1, be1.reshape(1, H), We2, be2.reshape(1, H))
    eaT = edge_attr.T.astype(BF16)
    eaE = eaT[:, 0::2]
    eaO = eaT[:, 1::2]
    qs = [_q_layer(eaE, eaO, Wm1[i, 2 * H:, :].astype(BF16))
          for i in range(DEPTH)]
    cnt16 = _cnt_kernel(dstr2, zc)

    for i in range(DEPTH):
        pa, pb = _pre(h, Wm1[i, :H, :], Wm1[i, H:2 * H, :],
                      bm1[i].reshape(1, H))
        pa2 = pa.reshape(2 * N, H // 4)
        pb2 = pb.reshape(2 * N, H // 4)
        S = _sc_layers[i](pa2, pb2, qs[i], gsrc, gdst, dstr, zrow)
        h = _update(S, cnt16, h, Wm2p[i], bm2[i].reshape(1, H),
                    Wu1[i, :H, :], Wu1[i, H:, :], bu1[i].reshape(1, H),
                    Wu2[i], bu2[i].reshape(1, H))

    return _pool_head(h, bidr, Wh1, bh1.reshape(1, H), Wh2, bh2.reshape(1, OUT))


# speedup vs baseline: 2.0267x; 1.3634x over previous
import functools

import numpy as np

import jax
import jax.numpy as jnp
from jax import lax
from jax.experimental import pallas as pl
from jax.experimental.pallas import tpu as pltpu
from jax.experimental.pallas import tpu_sc as plsc

N = 10000
E = 160000
D = 256
DE = 16
H = 256
OUT = 128
DEPTH = 3
G = 64

NC = 2
NS = 16
EPT = E // NS
ROWS_PT = N // NS
K = 50
SK = 1000
CPS = SK // K
NSUP = EPT // SK
ER = E // K

BN = 2000
BE = 640
F32 = jnp.float32
BF16 = jnp.bfloat16


def _permute_wm2(Wm2):
    w = Wm2.reshape(DEPTH, 2, 2, 4, 16, H)
    return jnp.transpose(w, (0, 1, 3, 2, 4, 5)).reshape(DEPTH, H, H)


def _pack_bf16_pairs(y):
    lo = jax.lax.bitcast_convert_type(
        y[:, :H // 4].astype(BF16), jnp.int16).astype(jnp.int32) & 0xFFFF
    hi = jax.lax.bitcast_convert_type(
        y[:, H // 4:].astype(BF16), jnp.int16).astype(jnp.int32) << 16
    return jax.lax.bitcast_convert_type(lo | hi, F32)


def _embed_body(x_ref, w1_ref, b1_ref, w2_ref, b2_ref, o_ref):
    h = jnp.maximum(
        jnp.dot(x_ref[...], w1_ref[...], preferred_element_type=F32) + b1_ref[0],
        0.0)
    o_ref[...] = jnp.dot(h, w2_ref[...], preferred_element_type=F32) + b2_ref[0]


def _embed(x, W1, b1, W2, b2):
    return pl.pallas_call(
        _embed_body,
        grid=(N // BN,),
        in_specs=[
            pl.BlockSpec((BN, D), lambda i: (i, 0)),
            pl.BlockSpec((D, H), lambda i: (0, 0)),
            pl.BlockSpec((1, H), lambda i: (0, 0)),
            pl.BlockSpec((H, H), lambda i: (0, 0)),
            pl.BlockSpec((1, H), lambda i: (0, 0)),
        ],
        out_specs=pl.BlockSpec((BN, H), lambda i: (i, 0)),
        out_shape=jax.ShapeDtypeStruct((N, H), F32),
    )(x, W1, b1, W2, b2)


def _q_body(eae_ref, eao_ref, wc_ref, q_ref):
    qe = jax.lax.dot_general(eae_ref[...], wc_ref[...],
                             (((0,), (0,)), ((), ())),
                             preferred_element_type=F32)
    qo = jax.lax.dot_general(eao_ref[...], wc_ref[...],
                             (((0,), (0,)), ((), ())),
                             preferred_element_type=F32)
    q_ref[...] = jnp.concatenate(
        [_pack_bf16_pairs(qe), _pack_bf16_pairs(qo)], axis=1)[None]


def _q_layer(eaE, eaO, wc):
    return pl.pallas_call(
        _q_body,
        grid=(2, E // 2 // BE),
        in_specs=[
            pl.BlockSpec((DE, BE), lambda c, e: (0, e)),
            pl.BlockSpec((DE, BE), lambda c, e: (0, e)),
            pl.BlockSpec((DE, H // 2), lambda c, e: (0, c)),
        ],
        out_specs=pl.BlockSpec((1, BE, H // 2), lambda c, e: (c, e, 0)),
        out_shape=jax.ShapeDtypeStruct((2, E // 2, H // 2), F32),
    )(eaE, eaO, wc)


def _pre_body(x_ref, wa_ref, wb_ref, bm_ref, pa_ref, pb_ref):
    xb = x_ref[...]
    for c in range(2):
        wc = pl.ds(c * (H // 2), H // 2)
        pa_ref[c] = _pack_bf16_pairs(
            jnp.dot(xb, wa_ref[:, wc], preferred_element_type=F32))
        pb_ref[c] = _pack_bf16_pairs(
            jnp.dot(xb, wb_ref[:, wc], preferred_element_type=F32)
            + bm_ref[0, wc])


def _pre(x, Wa, Wb, bm):
    return pl.pallas_call(
        _pre_body,
        grid=(N // BN,),
        in_specs=[
            pl.BlockSpec((BN, H), lambda i: (i, 0)),
            pl.BlockSpec((H, H), lambda i: (0, 0)),
            pl.BlockSpec((H, H), lambda i: (0, 0)),
            pl.BlockSpec((1, H), lambda i: (0, 0)),
        ],
        out_specs=[
            pl.BlockSpec((2, BN, H // 4), lambda i: (0, i, 0)),
            pl.BlockSpec((2, BN, H // 4), lambda i: (0, i, 0)),
        ],
        out_shape=[
            jax.ShapeDtypeStruct((2, N, H // 4), F32),
            jax.ShapeDtypeStruct((2, N, H // 4), F32),
        ],
    )(x, Wa, Wb, bm)


def _upd_body(s_ref, cnt_ref, x_ref, wm2_ref, bm2_ref, wua_ref, wub_ref,
              bu1_ref, wu2_ref, bu2_ref, o_ref):
    s0 = s_ref[0]
    s1 = s_ref[1]
    ssum = (jnp.dot(s0, wm2_ref[0:128, :], preferred_element_type=F32)
            + jnp.dot(s1, wm2_ref[128:256, :], preferred_element_type=F32))
    cnt = (cnt_ref[0] + cnt_ref[1])[:, 0:1]
    aggr = (ssum + cnt * bm2_ref[0]) / jnp.maximum(cnt, 1.0)
    xb = x_ref[...]
    h = jnp.maximum(
        jnp.dot(xb, wua_ref[...], preferred_element_type=F32)
        + jnp.dot(aggr, wub_ref[...], preferred_element_type=F32)
        + bu1_ref[0], 0.0)
    o_ref[...] = jnp.dot(h, wu2_ref[...], preferred_element_type=F32) + bu2_ref[0]


def _update(S, cnt16, x, Wm2i, bm2i, Wua, Wub, bu1i, Wu2i, bu2i):
    return pl.pallas_call(
        _upd_body,
        grid=(N // BN,),
        in_specs=[
            pl.BlockSpec((2, BN, H // 2), lambda i: (0, i, 0)),
            pl.BlockSpec((NC, BN, 16), lambda i: (0, i, 0)),
            pl.BlockSpec((BN, H), lambda i: (i, 0)),
            pl.BlockSpec((H, H), lambda i: (0, 0)),
            pl.BlockSpec((1, H), lambda i: (0, 0)),
            pl.BlockSpec((H, H), lambda i: (0, 0)),
            pl.BlockSpec((H, H), lambda i: (0, 0)),
            pl.BlockSpec((1, H), lambda i: (0, 0)),
            pl.BlockSpec((H, H), lambda i: (0, 0)),
            pl.BlockSpec((1, H), lambda i: (0, 0)),
        ],
        out_specs=pl.BlockSpec((BN, H), lambda i: (i, 0)),
        out_shape=jax.ShapeDtypeStruct((N, H), F32),
    )(S, cnt16, x, Wm2i, bm2i, Wua, Wub, bu1i, Wu2i, bu2i)


def _pool_body(x_ref, bid_ref, wh1_ref, bh1_ref, wh2_ref, bh2_ref, o_ref,
               acc_ref):
    i = pl.program_id(0)

    @pl.when(i == 0)
    def _init():
        acc_ref[...] = jnp.zeros_like(acc_ref)

    bid = bid_ref[0, 0]
    oh = (lax.broadcasted_iota(jnp.int32, (G, BN), 0)
          == bid[None, :]).astype(F32)
    acc_ref[...] += jnp.dot(oh, x_ref[...], preferred_element_type=F32)

    @pl.when(i == pl.num_programs(0) - 1)
    def _fin():
        h = jnp.maximum(
            jnp.dot(acc_ref[...], wh1_ref[...], preferred_element_type=F32)
            + bh1_ref[0], 0.0)
        o_ref[...] = jnp.dot(h, wh2_ref[...], preferred_element_type=F32) + bh2_ref[0]


def _pool_head(x, bidr, Wh1, bh1, Wh2, bh2):
    return pl.pallas_call(
        _pool_body,
        grid=(N // BN,),
        in_specs=[
            pl.BlockSpec((BN, H), lambda i: (i, 0)),
            pl.BlockSpec((1, 1, BN), lambda i: (i, 0, 0)),
            pl.BlockSpec((H, H), lambda i: (0, 0)),
            pl.BlockSpec((1, H), lambda i: (0, 0)),
            pl.BlockSpec((H, OUT), lambda i: (0, 0)),
            pl.BlockSpec((1, OUT), lambda i: (0, 0)),
        ],
        out_specs=pl.BlockSpec((G, OUT), lambda i: (0, 0)),
        out_shape=jax.ShapeDtypeStruct((G, OUT), F32),
        scratch_shapes=[pltpu.VMEM((G, H), F32)],
    )(x, bidr, Wh1, bh1, Wh2, bh2)


_MESH = plsc.VectorSubcoreMesh(core_axis_name="c", subcore_axis_name="s",
                               num_cores=NC, num_subcores=NS)
_SC_PARAMS = pltpu.CompilerParams(use_tc_tiling_on_sc=False,
                                  needs_layout_passes=False)


_KC = 100
_CROWS = E // _KC // (NC * NS)


def _cnt_body(dstr2_hbm, zc_hbm, cnt_out, sdst, vones, c_sh):
    cid = lax.axis_index("c")
    sid = lax.axis_index("s")
    myrows = pl.ds(sid * ROWS_PT, ROWS_PT)

    pltpu.sync_copy(zc_hbm.at[myrows], c_sh.at[myrows])

    def _ones_row(r, carry):
        vones[r] = jnp.ones((16,), F32)
        return carry
    lax.fori_loop(0, _KC, _ones_row, 0)
    plsc.subcore_barrier()

    base = (cid * NS + sid) * _CROWS
    pltpu.sync_copy(dstr2_hbm.at[pl.ds(base, _CROWS)], sdst)

    def chunk(c, carry):
        pltpu.sync_copy(vones, c_sh.at[sdst.at[c]], add=True)
        return carry
    lax.fori_loop(0, _CROWS, chunk, 0)
    plsc.subcore_barrier()

    pltpu.sync_copy(c_sh.at[myrows], cnt_out.at[cid, myrows])


_cnt_kernel = pl.kernel(
    _cnt_body,
    out_type=jax.ShapeDtypeStruct((NC, N, 16), F32),
    mesh=_MESH,
    scratch_types=[
        pltpu.VMEM((_CROWS, _KC), jnp.int32),
        pltpu.VMEM((_KC, 16), F32),
        pltpu.VMEM_SHARED((N, 16), F32),
    ],
    compiler_params=_SC_PARAMS)


def _make_sc(layer):
    scratch = [
        pltpu.VMEM((CPS, K), jnp.int32),
        pltpu.VMEM((CPS, K), jnp.int32),
        pltpu.VMEM((CPS, K), jnp.int32),
        pltpu.VMEM((2, K, H // 4), F32),
        pltpu.VMEM((2, K, H // 4), F32),
        pltpu.VMEM((2, K // 2, H // 2), F32),
        pltpu.VMEM((2, K, H // 2), F32),
        pltpu.VMEM_SHARED((N, H // 2), F32),
        pltpu.SemaphoreType.DMA,
        pltpu.SemaphoreType.DMA,
        pltpu.SemaphoreType.DMA,
    ]

    def body(pa_hbm, pb_hbm, ql_hbm, gsrc_hbm, gdst_hbm, dstr_hbm, z_hbm,
             s_out, isrc, idst, sdst, va, vb, vq, stg, s_sh, sem1, sem2,
             sem3):
        cid = lax.axis_index("c")
        sid = lax.axis_index("s")
        myrows = pl.ds(sid * ROWS_PT, ROWS_PT)

        pltpu.sync_copy(z_hbm.at[myrows], s_sh.at[myrows])
        plsc.subcore_barrier()

        def fire(sup_base_e, b):
            buf = b % 2
            cps = [
                pltpu.async_copy(pa_hbm.at[isrc.at[b]], va.at[buf], sem1),
                pltpu.async_copy(pb_hbm.at[idst.at[b]], vb.at[buf], sem1),
                pltpu.async_copy(
                    ql_hbm.at[cid, pl.ds((sup_base_e + b * K) // 2, K // 2)],
                    vq.at[buf], sem2),
            ]
            return cps

        def crunch(b):
            buf = b % 2

            @plsc.parallel_loop(0, K // 2, unroll=2)
            def rowf(r2):
                for p in range(2):
                    r = 2 * r2 + p
                    for g in range(H // 2 // 32):
                        sl = pl.ds(g * 16, 16)
                        a32 = plsc.bitcast(va[buf, r, sl], BF16)
                        b32 = plsc.bitcast(vb[buf, r, sl], BF16)
                        q32 = plsc.bitcast(
                            vq[buf, r2, pl.ds(p * 64 + g * 16, 16)], BF16)
                        v = jnp.maximum(a32 + b32 + q32,
                                        jnp.zeros((32,), BF16))
                        lo, hi = plsc.unpack(
                            v, format=plsc.PackFormat.INTERLEAVED)
                        stg[buf, r, pl.ds(g * 32, 16)] = lo
                        stg[buf, r, pl.ds(g * 32 + 16, 16)] = hi

        def super_loop(s, carry):
            base_row = sid * (EPT // K) + s * CPS
            base_e = sid * EPT + s * SK
            pltpu.sync_copy(gsrc_hbm.at[cid, pl.ds(base_row, CPS)], isrc)
            pltpu.sync_copy(gdst_hbm.at[cid, pl.ds(base_row, CPS)], idst)
            pltpu.sync_copy(dstr_hbm.at[pl.ds(base_row, CPS)], sdst)
            cps = fire(base_e, 0)
            scats = [None, None]
            for b in range(CPS):
                for cp in cps:
                    cp.wait()
                if b + 1 < CPS:
                    cps = fire(base_e, b + 1)
                if scats[b % 2] is not None:
                    scats[b % 2].wait()
                crunch(b)
                scats[b % 2] = pltpu.async_copy(
                    stg.at[b % 2], s_sh.at[sdst.at[b]], sem3, add=True)
            scats[0].wait()
            scats[1].wait()
            return carry
        lax.fori_loop(0, NSUP, super_loop, 0)
        plsc.subcore_barrier()

        pltpu.sync_copy(s_sh.at[myrows], s_out.at[cid, myrows])

    return pl.kernel(body,
                     out_type=jax.ShapeDtypeStruct((2, N, H // 2), F32),
                     mesh=_MESH, scratch_types=scratch,
                     compiler_params=_SC_PARAMS)


_sc_layers = [_make_sc(i) for i in range(DEPTH)]


def kernel(x, edge_index, edge_attr, batch_ids, We1, be1, We2, be2,
           Wm1, bm1, Wm2, bm2, Wu1, bu1, Wu2, bu2, Wh1, bh1, Wh2, bh2):
    src = edge_index[0].astype(jnp.int32)
    dst = edge_index[1].astype(jnp.int32)
    gsrc = jnp.stack([src, N + src]).reshape(2, ER, K)
    gdst = jnp.stack([dst, N + dst]).reshape(2, ER, K)
    dstr = dst.reshape(ER, K)
    dstr2 = dst.reshape(E // _KC, _KC)
    zrow = jnp.zeros((N, H // 2), F32)
    zc = jnp.zeros((N, 16), F32)
    bidr = batch_ids.astype(jnp.int32).reshape(N // BN, 1, BN)
    Wm2p = _permute_wm2(Wm2)

    h = _embed(x, We1, be1.reshape(1, H), We2, be2.reshape(1, H))
    eaT = edge_attr.T.astype(BF16)
    eaE = eaT[:, 0::2]
    eaO = eaT[:, 1::2]
    qs = [_q_layer(eaE, eaO, Wm1[i, 2 * H:, :].astype(BF16))
          for i in range(DEPTH)]
    cnt16 = _cnt_kernel(dstr2, zc)

    for i in range(DEPTH):
        pa, pb = _pre(h, Wm1[i, :H, :], Wm1[i, H:2 * H, :],
                      bm1[i].reshape(1, H))
        pa2 = pa.reshape(2 * N, H // 4)
        pb2 = pb.reshape(2 * N, H // 4)
        S = _sc_layers[i](pa2, pb2, qs[i], gsrc, gdst, dstr, zrow)
        h = _update(S, cnt16, h, Wm2p[i], bm2[i].reshape(1, H),
                    Wu1[i, :H, :], Wu1[i, H:, :], bu1[i].reshape(1, H),
                    Wu2[i], bu2[i].reshape(1, H))

    return _pool_head(h, bidr, Wh1, bh1.reshape(1, H), Wh2, bh2.reshape(1, OUT))


# speedup vs baseline: 2.0493x; 1.0111x over previous
import functools

import numpy as np

import jax
import jax.numpy as jnp
from jax import lax
from jax.experimental import pallas as pl
from jax.experimental.pallas import tpu as pltpu
from jax.experimental.pallas import tpu_sc as plsc

N = 10000
E = 160000
D = 256
DE = 16
H = 256
OUT = 128
DEPTH = 3
G = 64

NC = 2
NS = 16
EPT = E // NS
ROWS_PT = N // NS
K = 50
SK = 1000
CPS = SK // K
NSUP = EPT // SK
ER = E // K

BN = 2000
BE = 640
F32 = jnp.float32
BF16 = jnp.bfloat16


def _permute_wm2(Wm2):
    w = Wm2.reshape(DEPTH, 2, 2, 4, 16, H)
    return jnp.transpose(w, (0, 1, 3, 2, 4, 5)).reshape(DEPTH, H, H)


def _pack_bf16_pairs(y):
    lo = jax.lax.bitcast_convert_type(
        y[:, :H // 4].astype(BF16), jnp.int16).astype(jnp.int32) & 0xFFFF
    hi = jax.lax.bitcast_convert_type(
        y[:, H // 4:].astype(BF16), jnp.int16).astype(jnp.int32) << 16
    return jax.lax.bitcast_convert_type(lo | hi, F32)


def _embed_body(x_ref, w1_ref, b1_ref, w2_ref, b2_ref, o_ref):
    h = jnp.maximum(
        jnp.dot(x_ref[...], w1_ref[...], preferred_element_type=F32) + b1_ref[0],
        0.0)
    o_ref[...] = jnp.dot(h, w2_ref[...], preferred_element_type=F32) + b2_ref[0]


def _embed(x, W1, b1, W2, b2):
    return pl.pallas_call(
        _embed_body,
        grid=(N // BN,),
        in_specs=[
            pl.BlockSpec((BN, D), lambda i: (i, 0)),
            pl.BlockSpec((D, H), lambda i: (0, 0)),
            pl.BlockSpec((1, H), lambda i: (0, 0)),
            pl.BlockSpec((H, H), lambda i: (0, 0)),
            pl.BlockSpec((1, H), lambda i: (0, 0)),
        ],
        out_specs=pl.BlockSpec((BN, H), lambda i: (i, 0)),
        out_shape=jax.ShapeDtypeStruct((N, H), F32),
    )(x, W1, b1, W2, b2)


def _q_body(eae_ref, eao_ref, wc_ref, q_ref):
    qe = jax.lax.dot_general(eae_ref[...], wc_ref[...],
                             (((0,), (0,)), ((), ())),
                             preferred_element_type=F32)
    qo = jax.lax.dot_general(eao_ref[...], wc_ref[...],
                             (((0,), (0,)), ((), ())),
                             preferred_element_type=F32)
    q_ref[...] = jnp.concatenate(
        [_pack_bf16_pairs(qe), _pack_bf16_pairs(qo)], axis=1)[None]


def _q_layer(eaE, eaO, wc):
    return pl.pallas_call(
        _q_body,
        grid=(2, E // 2 // BE),
        in_specs=[
            pl.BlockSpec((DE, BE), lambda c, e: (0, e)),
            pl.BlockSpec((DE, BE), lambda c, e: (0, e)),
            pl.BlockSpec((DE, H // 2), lambda c, e: (0, c)),
        ],
        out_specs=pl.BlockSpec((1, BE, H // 2), lambda c, e: (c, e, 0)),
        out_shape=jax.ShapeDtypeStruct((2, E // 2, H // 2), F32),
    )(eaE, eaO, wc)


def _pre_body(x_ref, wa_ref, wb_ref, bm_ref, pa_ref, pb_ref):
    xb = x_ref[...]
    for c in range(2):
        wc = pl.ds(c * (H // 2), H // 2)
        pa_ref[c] = _pack_bf16_pairs(
            jnp.dot(xb, wa_ref[:, wc], preferred_element_type=F32))
        pb_ref[c] = _pack_bf16_pairs(
            jnp.dot(xb, wb_ref[:, wc], preferred_element_type=F32)
            + bm_ref[0, wc])


def _pre(x, Wa, Wb, bm):
    return pl.pallas_call(
        _pre_body,
        grid=(N // BN,),
        in_specs=[
            pl.BlockSpec((BN, H), lambda i: (i, 0)),
            pl.BlockSpec((H, H), lambda i: (0, 0)),
            pl.BlockSpec((H, H), lambda i: (0, 0)),
            pl.BlockSpec((1, H), lambda i: (0, 0)),
        ],
        out_specs=[
            pl.BlockSpec((2, BN, H // 4), lambda i: (0, i, 0)),
            pl.BlockSpec((2, BN, H // 4), lambda i: (0, i, 0)),
        ],
        out_shape=[
            jax.ShapeDtypeStruct((2, N, H // 4), F32),
            jax.ShapeDtypeStruct((2, N, H // 4), F32),
        ],
    )(x, Wa, Wb, bm)


def _upd_body(s_ref, cnt_ref, x_ref, wm2_ref, bm2_ref, wua_ref, wub_ref,
              bu1_ref, wu2_ref, bu2_ref, o_ref):
    s0 = s_ref[0]
    s1 = s_ref[1]
    ssum = (jnp.dot(s0, wm2_ref[0:128, :], preferred_element_type=F32)
            + jnp.dot(s1, wm2_ref[128:256, :], preferred_element_type=F32))
    cnt = (cnt_ref[0] + cnt_ref[1])[:, 0:1]
    aggr = (ssum + cnt * bm2_ref[0]) / jnp.maximum(cnt, 1.0)
    xb = x_ref[...]
    h = jnp.maximum(
        jnp.dot(xb, wua_ref[...], preferred_element_type=F32)
        + jnp.dot(aggr, wub_ref[...], preferred_element_type=F32)
        + bu1_ref[0], 0.0)
    o_ref[...] = jnp.dot(h, wu2_ref[...], preferred_element_type=F32) + bu2_ref[0]


def _update(S, cnt16, x, Wm2i, bm2i, Wua, Wub, bu1i, Wu2i, bu2i):
    return pl.pallas_call(
        _upd_body,
        grid=(N // BN,),
        in_specs=[
            pl.BlockSpec((2, BN, H // 2), lambda i: (0, i, 0)),
            pl.BlockSpec((NC, BN, 16), lambda i: (0, i, 0)),
            pl.BlockSpec((BN, H), lambda i: (i, 0)),
            pl.BlockSpec((H, H), lambda i: (0, 0)),
            pl.BlockSpec((1, H), lambda i: (0, 0)),
            pl.BlockSpec((H, H), lambda i: (0, 0)),
            pl.BlockSpec((H, H), lambda i: (0, 0)),
            pl.BlockSpec((1, H), lambda i: (0, 0)),
            pl.BlockSpec((H, H), lambda i: (0, 0)),
            pl.BlockSpec((1, H), lambda i: (0, 0)),
        ],
        out_specs=pl.BlockSpec((BN, H), lambda i: (i, 0)),
        out_shape=jax.ShapeDtypeStruct((N, H), F32),
    )(S, cnt16, x, Wm2i, bm2i, Wua, Wub, bu1i, Wu2i, bu2i)


def _pool_body(x_ref, bid_ref, wh1_ref, bh1_ref, wh2_ref, bh2_ref, o_ref,
               acc_ref):
    i = pl.program_id(0)

    @pl.when(i == 0)
    def _init():
        acc_ref[...] = jnp.zeros_like(acc_ref)

    bid = bid_ref[0, 0]
    oh = (lax.broadcasted_iota(jnp.int32, (G, BN), 0)
          == bid[None, :]).astype(F32)
    acc_ref[...] += jnp.dot(oh, x_ref[...], preferred_element_type=F32)

    @pl.when(i == pl.num_programs(0) - 1)
    def _fin():
        h = jnp.maximum(
            jnp.dot(acc_ref[...], wh1_ref[...], preferred_element_type=F32)
            + bh1_ref[0], 0.0)
        o_ref[...] = jnp.dot(h, wh2_ref[...], preferred_element_type=F32) + bh2_ref[0]


def _pool_head(x, bidr, Wh1, bh1, Wh2, bh2):
    return pl.pallas_call(
        _pool_body,
        grid=(N // BN,),
        in_specs=[
            pl.BlockSpec((BN, H), lambda i: (i, 0)),
            pl.BlockSpec((1, 1, BN), lambda i: (i, 0, 0)),
            pl.BlockSpec((H, H), lambda i: (0, 0)),
            pl.BlockSpec((1, H), lambda i: (0, 0)),
            pl.BlockSpec((H, OUT), lambda i: (0, 0)),
            pl.BlockSpec((1, OUT), lambda i: (0, 0)),
        ],
        out_specs=pl.BlockSpec((G, OUT), lambda i: (0, 0)),
        out_shape=jax.ShapeDtypeStruct((G, OUT), F32),
        scratch_shapes=[pltpu.VMEM((G, H), F32)],
    )(x, bidr, Wh1, bh1, Wh2, bh2)


_MESH = plsc.VectorSubcoreMesh(core_axis_name="c", subcore_axis_name="s",
                               num_cores=NC, num_subcores=NS)
_SC_PARAMS = pltpu.CompilerParams(use_tc_tiling_on_sc=False,
                                  needs_layout_passes=False)


_KC = 100
_CROWS = E // _KC // (NC * NS)


def _cnt_body(dstr2_hbm, zc_hbm, cnt_out, sdst, vones, c_sh):
    cid = lax.axis_index("c")
    sid = lax.axis_index("s")
    myrows = pl.ds(sid * ROWS_PT, ROWS_PT)

    pltpu.sync_copy(zc_hbm.at[myrows], c_sh.at[myrows])

    def _ones_row(r, carry):
        vones[r] = jnp.ones((16,), F32)
        return carry
    lax.fori_loop(0, _KC, _ones_row, 0)
    plsc.subcore_barrier()

    base = (cid * NS + sid) * _CROWS
    pltpu.sync_copy(dstr2_hbm.at[pl.ds(base, _CROWS)], sdst)

    def chunk(c, carry):
        pltpu.sync_copy(vones, c_sh.at[sdst.at[c]], add=True)
        return carry
    lax.fori_loop(0, _CROWS, chunk, 0)
    plsc.subcore_barrier()

    pltpu.sync_copy(c_sh.at[myrows], cnt_out.at[cid, myrows])


_cnt_kernel = pl.kernel(
    _cnt_body,
    out_type=jax.ShapeDtypeStruct((NC, N, 16), F32),
    mesh=_MESH,
    scratch_types=[
        pltpu.VMEM((_CROWS, _KC), jnp.int32),
        pltpu.VMEM((_KC, 16), F32),
        pltpu.VMEM_SHARED((N, 16), F32),
    ],
    compiler_params=_SC_PARAMS)


def _make_sc(layer):
    scratch = [
        pltpu.VMEM((CPS, K), jnp.int32),
        pltpu.VMEM((CPS, K), jnp.int32),
        pltpu.VMEM((CPS, K), jnp.int32),
        pltpu.VMEM((2, K, H // 4), F32),
        pltpu.VMEM((2, K, H // 4), F32),
        pltpu.VMEM((2, K // 2, H // 2), F32),
        pltpu.VMEM((2, K, H // 2), F32),
        pltpu.VMEM_SHARED((N, H // 2), F32),
        pltpu.SemaphoreType.DMA,
        pltpu.SemaphoreType.DMA,
        pltpu.SemaphoreType.DMA,
    ]

    def body(pa_hbm, pb_hbm, ql_hbm, gsrc_hbm, gdst_hbm, dstr_hbm, z_hbm,
             s_out, isrc, idst, sdst, va, vb, vq, stg, s_sh, sem1, sem2,
             sem3):
        cid = lax.axis_index("c")
        sid = lax.axis_index("s")
        myrows = pl.ds(sid * ROWS_PT, ROWS_PT)

        pltpu.sync_copy(z_hbm.at[myrows], s_sh.at[myrows])
        plsc.subcore_barrier()

        def fire(sup_base_e, b):
            buf = b % 2
            cps = [
                pltpu.async_copy(pa_hbm.at[isrc.at[b]], va.at[buf], sem1),
                pltpu.async_copy(pb_hbm.at[idst.at[b]], vb.at[buf], sem1),
                pltpu.async_copy(
                    ql_hbm.at[cid, pl.ds((sup_base_e + b * K) // 2, K // 2)],
                    vq.at[buf], sem2),
            ]
            return cps

        def crunch(b):
            buf = b % 2

            @plsc.parallel_loop(0, K // 2, unroll=5)
            def rowf(r2):
                for p in range(2):
                    r = 2 * r2 + p
                    for g in range(H // 2 // 32):
                        sl = pl.ds(g * 16, 16)
                        a32 = plsc.bitcast(va[buf, r, sl], BF16)
                        b32 = plsc.bitcast(vb[buf, r, sl], BF16)
                        q32 = plsc.bitcast(
                            vq[buf, r2, pl.ds(p * 64 + g * 16, 16)], BF16)
                        v = jnp.maximum(a32 + b32 + q32,
                                        jnp.zeros((32,), BF16))
                        lo, hi = plsc.unpack(
                            v, format=plsc.PackFormat.INTERLEAVED)
                        stg[buf, r, pl.ds(g * 32, 16)] = lo
                        stg[buf, r, pl.ds(g * 32 + 16, 16)] = hi

        def super_loop(s, carry):
            base_row = sid * (EPT // K) + s * CPS
            base_e = sid * EPT + s * SK
            pltpu.sync_copy(gsrc_hbm.at[cid, pl.ds(base_row, CPS)], isrc)
            pltpu.sync_copy(gdst_hbm.at[cid, pl.ds(base_row, CPS)], idst)
            pltpu.sync_copy(dstr_hbm.at[pl.ds(base_row, CPS)], sdst)
            cps = fire(base_e, 0)
            scats = [None, None]
            for b in range(CPS):
                for cp in cps:
                    cp.wait()
                if b + 1 < CPS:
                    cps = fire(base_e, b + 1)
                if scats[b % 2] is not None:
                    scats[b % 2].wait()
                crunch(b)
                scats[b % 2] = pltpu.async_copy(
                    stg.at[b % 2], s_sh.at[sdst.at[b]], sem3, add=True)
            scats[0].wait()
            scats[1].wait()
            return carry
        lax.fori_loop(0, NSUP, super_loop, 0)
        plsc.subcore_barrier()

        pltpu.sync_copy(s_sh.at[myrows], s_out.at[cid, myrows])

    return pl.kernel(body,
                     out_type=jax.ShapeDtypeStruct((2, N, H // 2), F32),
                     mesh=_MESH, scratch_types=scratch,
                     compiler_params=_SC_PARAMS)


_sc_layers = [_make_sc(i) for i in range(DEPTH)]


def kernel(x, edge_index, edge_attr, batch_ids, We1, be1, We2, be2,
           Wm1, bm1, Wm2, bm2, Wu1, bu1, Wu2, bu2, Wh1, bh1, Wh2, bh2):
    src = edge_index[0].astype(jnp.int32)
    dst = edge_index[1].astype(jnp.int32)
    gsrc = jnp.stack([src, N + src]).reshape(2, ER, K)
    gdst = jnp.stack([dst, N + dst]).reshape(2, ER, K)
    dstr = dst.reshape(ER, K)
    dstr2 = dst.reshape(E // _KC, _KC)
    zrow = jnp.zeros((N, H // 2), F32)
    zc = jnp.zeros((N, 16), F32)
    bidr = batch_ids.astype(jnp.int32).reshape(N // BN, 1, BN)
    Wm2p = _permute_wm2(Wm2)

    h = _embed(x, We1, be1.reshape(1, H), We2, be2.reshape(1, H))
    eaT = edge_attr.T.astype(BF16)
    eaE = eaT[:, 0::2]
    eaO = eaT[:, 1::2]
    qs = [_q_layer(eaE, eaO, Wm1[i, 2 * H:, :].astype(BF16))
          for i in range(DEPTH)]
    cnt16 = _cnt_kernel(dstr2, zc)

    for i in range(DEPTH):
        pa, pb = _pre(h, Wm1[i, :H, :], Wm1[i, H:2 * H, :],
                      bm1[i].reshape(1, H))
        pa2 = pa.reshape(2 * N, H // 4)
        pb2 = pb.reshape(2 * N, H // 4)
        S = _sc_layers[i](pa2, pb2, qs[i], gsrc, gdst, dstr, zrow)
        h = _update(S, cnt16, h, Wm2p[i], bm2[i].reshape(1, H),
                    Wu1[i, :H, :], Wu1[i, H:, :], bu1[i].reshape(1, H),
                    Wu2[i], bu2[i].reshape(1, H))

    return _pool_head(h, bidr, Wh1, bh1.reshape(1, H), Wh2, bh2.reshape(1, OUT))


# speedup vs baseline: 2.0941x; 1.0218x over previous
import functools

import numpy as np

import jax
import jax.numpy as jnp
from jax import lax
from jax.experimental import pallas as pl
from jax.experimental.pallas import tpu as pltpu
from jax.experimental.pallas import tpu_sc as plsc

N = 10000
E = 160000
D = 256
DE = 16
H = 256
OUT = 128
DEPTH = 3
G = 64

NC = 2
NS = 16
EPT = E // NS
ROWS_PT = N // NS
K = 50
SK = 1000
CPS = SK // K
NSUP = EPT // SK
ER = E // K

BN = 2000
BE = 640
F32 = jnp.float32
BF16 = jnp.bfloat16


def _permute_wm2(Wm2):
    w = Wm2.reshape(DEPTH, 2, 2, 4, 16, H)
    return jnp.transpose(w, (0, 1, 3, 2, 4, 5)).reshape(DEPTH, H, H)


def _pack_bf16_pairs(y):
    lo = jax.lax.bitcast_convert_type(
        y[:, :H // 4].astype(BF16), jnp.int16).astype(jnp.int32) & 0xFFFF
    hi = jax.lax.bitcast_convert_type(
        y[:, H // 4:].astype(BF16), jnp.int16).astype(jnp.int32) << 16
    return jax.lax.bitcast_convert_type(lo | hi, F32)


def _pack_pre(h, wa_ref, wb_ref, bm_ref, pa_ref, pb_ref):
    for c in range(2):
        wc = pl.ds(c * (H // 2), H // 2)
        pa_ref[c] = _pack_bf16_pairs(
            jnp.dot(h, wa_ref[:, wc], preferred_element_type=F32))
        pb_ref[c] = _pack_bf16_pairs(
            jnp.dot(h, wb_ref[:, wc], preferred_element_type=F32)
            + bm_ref[0, wc])


def _embed_pre_body(x_ref, w1_ref, b1_ref, w2_ref, b2_ref, wa_ref, wb_ref,
                    bm_ref, h_ref, pa_ref, pb_ref):
    h = jnp.maximum(
        jnp.dot(x_ref[...], w1_ref[...], preferred_element_type=F32) + b1_ref[0],
        0.0)
    h = jnp.dot(h, w2_ref[...], preferred_element_type=F32) + b2_ref[0]
    h_ref[...] = h
    _pack_pre(h, wa_ref, wb_ref, bm_ref, pa_ref, pb_ref)


_W_SPEC = pl.BlockSpec((H, H), lambda i: (0, 0))
_B_SPEC = pl.BlockSpec((1, H), lambda i: (0, 0))
_PK_SPEC = pl.BlockSpec((2, BN, H // 4), lambda i: (0, i, 0))
_PK_SHAPE = jax.ShapeDtypeStruct((2, N, H // 4), F32)


def _embed_pre(x, W1, b1, W2, b2, Wa, Wb, bm):
    return pl.pallas_call(
        _embed_pre_body,
        grid=(N // BN,),
        in_specs=[pl.BlockSpec((BN, D), lambda i: (i, 0)),
                  _W_SPEC, _B_SPEC, _W_SPEC, _B_SPEC,
                  _W_SPEC, _W_SPEC, _B_SPEC],
        out_specs=[pl.BlockSpec((BN, H), lambda i: (i, 0)),
                   _PK_SPEC, _PK_SPEC],
        out_shape=[jax.ShapeDtypeStruct((N, H), F32), _PK_SHAPE, _PK_SHAPE],
    )(x, W1, b1, W2, b2, Wa, Wb, bm)


def _q_body(eae_ref, eao_ref, wc_ref, q_ref):
    qe = jax.lax.dot_general(eae_ref[...], wc_ref[...],
                             (((0,), (0,)), ((), ())),
                             preferred_element_type=F32)
    qo = jax.lax.dot_general(eao_ref[...], wc_ref[...],
                             (((0,), (0,)), ((), ())),
                             preferred_element_type=F32)
    q_ref[...] = jnp.concatenate(
        [_pack_bf16_pairs(qe), _pack_bf16_pairs(qo)], axis=1)[None]


def _q_layer(eaE, eaO, wc):
    return pl.pallas_call(
        _q_body,
        grid=(2, E // 2 // BE),
        in_specs=[
            pl.BlockSpec((DE, BE), lambda c, e: (0, e)),
            pl.BlockSpec((DE, BE), lambda c, e: (0, e)),
            pl.BlockSpec((DE, H // 2), lambda c, e: (0, c)),
        ],
        out_specs=pl.BlockSpec((1, BE, H // 2), lambda c, e: (c, e, 0)),
        out_shape=jax.ShapeDtypeStruct((2, E // 2, H // 2), F32),
    )(eaE, eaO, wc)


def _upd_core(s_ref, cnt_ref, x_ref, wm2_ref, bm2_ref, wua_ref, wub_ref,
              bu1_ref, wu2_ref, bu2_ref):
    s0 = s_ref[0]
    s1 = s_ref[1]
    ssum = (jnp.dot(s0, wm2_ref[0:128, :], preferred_element_type=F32)
            + jnp.dot(s1, wm2_ref[128:256, :], preferred_element_type=F32))
    cnt = (cnt_ref[0] + cnt_ref[1])[:, 0:1]
    aggr = (ssum + cnt * bm2_ref[0]) / jnp.maximum(cnt, 1.0)
    xb = x_ref[...]
    h = jnp.maximum(
        jnp.dot(xb, wua_ref[...], preferred_element_type=F32)
        + jnp.dot(aggr, wub_ref[...], preferred_element_type=F32)
        + bu1_ref[0], 0.0)
    return jnp.dot(h, wu2_ref[...], preferred_element_type=F32) + bu2_ref[0]


def _upd_pre_body(s_ref, cnt_ref, x_ref, wm2_ref, bm2_ref, wua_ref, wub_ref,
                  bu1_ref, wu2_ref, bu2_ref, wa_ref, wb_ref, bm_ref,
                  h_ref, pa_ref, pb_ref):
    xn = _upd_core(s_ref, cnt_ref, x_ref, wm2_ref, bm2_ref, wua_ref, wub_ref,
                   bu1_ref, wu2_ref, bu2_ref)
    h_ref[...] = xn
    _pack_pre(xn, wa_ref, wb_ref, bm_ref, pa_ref, pb_ref)


_UPD_IN_SPECS = [
    pl.BlockSpec((2, BN, H // 2), lambda i: (0, i, 0)),
    pl.BlockSpec((NC, BN, 16), lambda i: (0, i, 0)),
    pl.BlockSpec((BN, H), lambda i: (i, 0)),
    _W_SPEC, _B_SPEC, _W_SPEC, _W_SPEC, _B_SPEC, _W_SPEC, _B_SPEC,
]


def _update_pre(S, cnt16, x, Wm2i, bm2i, Wua, Wub, bu1i, Wu2i, bu2i,
                Wa, Wb, bm):
    return pl.pallas_call(
        _upd_pre_body,
        grid=(N // BN,),
        in_specs=_UPD_IN_SPECS + [_W_SPEC, _W_SPEC, _B_SPEC],
        out_specs=[pl.BlockSpec((BN, H), lambda i: (i, 0)),
                   _PK_SPEC, _PK_SPEC],
        out_shape=[jax.ShapeDtypeStruct((N, H), F32), _PK_SHAPE, _PK_SHAPE],
    )(S, cnt16, x, Wm2i, bm2i, Wua, Wub, bu1i, Wu2i, bu2i, Wa, Wb, bm)


def _upd_pool_body(s_ref, cnt_ref, x_ref, wm2_ref, bm2_ref, wua_ref, wub_ref,
                   bu1_ref, wu2_ref, bu2_ref, bid_ref, wh1_ref, bh1_ref,
                   wh2_ref, bh2_ref, o_ref, acc_ref):
    i = pl.program_id(0)

    @pl.when(i == 0)
    def _init():
        acc_ref[...] = jnp.zeros_like(acc_ref)

    xn = _upd_core(s_ref, cnt_ref, x_ref, wm2_ref, bm2_ref, wua_ref, wub_ref,
                   bu1_ref, wu2_ref, bu2_ref)
    bid = bid_ref[0, 0]
    oh = (lax.broadcasted_iota(jnp.int32, (G, BN), 0)
          == bid[None, :]).astype(F32)
    acc_ref[...] += jnp.dot(oh, xn, preferred_element_type=F32)

    @pl.when(i == pl.num_programs(0) - 1)
    def _fin():
        h = jnp.maximum(
            jnp.dot(acc_ref[...], wh1_ref[...], preferred_element_type=F32)
            + bh1_ref[0], 0.0)
        o_ref[...] = jnp.dot(h, wh2_ref[...], preferred_element_type=F32) + bh2_ref[0]


def _update_pool(S, cnt16, x, Wm2i, bm2i, Wua, Wub, bu1i, Wu2i, bu2i,
                 bidr, Wh1, bh1, Wh2, bh2):
    return pl.pallas_call(
        _upd_pool_body,
        grid=(N // BN,),
        in_specs=_UPD_IN_SPECS + [
            pl.BlockSpec((1, 1, BN), lambda i: (i, 0, 0)),
            _W_SPEC, _B_SPEC,
            pl.BlockSpec((H, OUT), lambda i: (0, 0)),
            pl.BlockSpec((1, OUT), lambda i: (0, 0)),
        ],
        out_specs=pl.BlockSpec((G, OUT), lambda i: (0, 0)),
        out_shape=jax.ShapeDtypeStruct((G, OUT), F32),
        scratch_shapes=[pltpu.VMEM((G, H), F32)],
    )(S, cnt16, x, Wm2i, bm2i, Wua, Wub, bu1i, Wu2i, bu2i,
      bidr, Wh1, bh1, Wh2, bh2)


_MESH = plsc.VectorSubcoreMesh(core_axis_name="c", subcore_axis_name="s",
                               num_cores=NC, num_subcores=NS)
_SC_PARAMS = pltpu.CompilerParams(use_tc_tiling_on_sc=False,
                                  needs_layout_passes=False)


_KC = 100
_CROWS = E // _KC // (NC * NS)


def _cnt_body(dstr2_hbm, zc_hbm, cnt_out, sdst, vones, c_sh):
    cid = lax.axis_index("c")
    sid = lax.axis_index("s")
    myrows = pl.ds(sid * ROWS_PT, ROWS_PT)

    pltpu.sync_copy(zc_hbm.at[myrows], c_sh.at[myrows])

    def _ones_row(r, carry):
        vones[r] = jnp.ones((16,), F32)
        return carry
    lax.fori_loop(0, _KC, _ones_row, 0)
    plsc.subcore_barrier()

    base = (cid * NS + sid) * _CROWS
    pltpu.sync_copy(dstr2_hbm.at[pl.ds(base, _CROWS)], sdst)

    def chunk(c, carry):
        pltpu.sync_copy(vones, c_sh.at[sdst.at[c]], add=True)
        return carry
    lax.fori_loop(0, _CROWS, chunk, 0)
    plsc.subcore_barrier()

    pltpu.sync_copy(c_sh.at[myrows], cnt_out.at[cid, myrows])


_cnt_kernel = pl.kernel(
    _cnt_body,
    out_type=jax.ShapeDtypeStruct((NC, N, 16), F32),
    mesh=_MESH,
    scratch_types=[
        pltpu.VMEM((_CROWS, _KC), jnp.int32),
        pltpu.VMEM((_KC, 16), F32),
        pltpu.VMEM_SHARED((N, 16), F32),
    ],
    compiler_params=_SC_PARAMS)


def _make_sc(layer):
    scratch = [
        pltpu.VMEM((CPS, K), jnp.int32),
        pltpu.VMEM((CPS, K), jnp.int32),
        pltpu.VMEM((CPS, K), jnp.int32),
        pltpu.VMEM((2, K, H // 4), F32),
        pltpu.VMEM((2, K, H // 4), F32),
        pltpu.VMEM((2, K // 2, H // 2), F32),
        pltpu.VMEM((2, K, H // 2), F32),
        pltpu.VMEM_SHARED((N, H // 2), F32),
        pltpu.SemaphoreType.DMA,
        pltpu.SemaphoreType.DMA,
        pltpu.SemaphoreType.DMA,
    ]

    def body(pa_hbm, pb_hbm, ql_hbm, gsrc_hbm, gdst_hbm, dstr_hbm, z_hbm,
             s_out, isrc, idst, sdst, va, vb, vq, stg, s_sh, sem1, sem2,
             sem3):
        cid = lax.axis_index("c")
        sid = lax.axis_index("s")
        myrows = pl.ds(sid * ROWS_PT, ROWS_PT)

        pltpu.sync_copy(z_hbm.at[myrows], s_sh.at[myrows])
        plsc.subcore_barrier()

        def fire(sup_base_e, b):
            buf = b % 2
            cps = [
                pltpu.async_copy(pa_hbm.at[isrc.at[b]], va.at[buf], sem1),
                pltpu.async_copy(pb_hbm.at[idst.at[b]], vb.at[buf], sem1),
                pltpu.async_copy(
                    ql_hbm.at[cid, pl.ds((sup_base_e + b * K) // 2, K // 2)],
                    vq.at[buf], sem2),
            ]
            return cps

        def crunch(b):
            buf = b % 2

            @plsc.parallel_loop(0, K // 2, unroll=5)
            def rowf(r2):
                for p in range(2):
                    r = 2 * r2 + p
                    for g in range(H // 2 // 32):
                        sl = pl.ds(g * 16, 16)
                        a32 = plsc.bitcast(va[buf, r, sl], BF16)
                        b32 = plsc.bitcast(vb[buf, r, sl], BF16)
                        q32 = plsc.bitcast(
                            vq[buf, r2, pl.ds(p * 64 + g * 16, 16)], BF16)
                        v = jnp.maximum(a32 + b32 + q32,
                                        jnp.zeros((32,), BF16))
                        lo, hi = plsc.unpack(
                            v, format=plsc.PackFormat.INTERLEAVED)
                        stg[buf, r, pl.ds(g * 32, 16)] = lo
                        stg[buf, r, pl.ds(g * 32 + 16, 16)] = hi

        def super_loop(s, carry):
            base_row = sid * (EPT // K) + s * CPS
            base_e = sid * EPT + s * SK
            pltpu.sync_copy(gsrc_hbm.at[cid, pl.ds(base_row, CPS)], isrc)
            pltpu.sync_copy(gdst_hbm.at[cid, pl.ds(base_row, CPS)], idst)
            pltpu.sync_copy(dstr_hbm.at[pl.ds(base_row, CPS)], sdst)
            cps = fire(base_e, 0)
            scats = [None, None]
            for b in range(CPS):
                for cp in cps:
                    cp.wait()
                if b + 1 < CPS:
                    cps = fire(base_e, b + 1)
                if scats[b % 2] is not None:
                    scats[b % 2].wait()
                crunch(b)
                scats[b % 2] = pltpu.async_copy(
                    stg.at[b % 2], s_sh.at[sdst.at[b]], sem3, add=True)
            scats[0].wait()
            scats[1].wait()
            return carry
        lax.fori_loop(0, NSUP, super_loop, 0)
        plsc.subcore_barrier()

        pltpu.sync_copy(s_sh.at[myrows], s_out.at[cid, myrows])

    return pl.kernel(body,
                     out_type=jax.ShapeDtypeStruct((2, N, H // 2), F32),
                     mesh=_MESH, scratch_types=scratch,
                     compiler_params=_SC_PARAMS)


_sc_layers = [_make_sc(i) for i in range(DEPTH)]


def kernel(x, edge_index, edge_attr, batch_ids, We1, be1, We2, be2,
           Wm1, bm1, Wm2, bm2, Wu1, bu1, Wu2, bu2, Wh1, bh1, Wh2, bh2):
    src = edge_index[0].astype(jnp.int32)
    dst = edge_index[1].astype(jnp.int32)
    gsrc = jnp.stack([src, N + src]).reshape(2, ER, K)
    gdst = jnp.stack([dst, N + dst]).reshape(2, ER, K)
    dstr = dst.reshape(ER, K)
    dstr2 = dst.reshape(E // _KC, _KC)
    zrow = jnp.zeros((N, H // 2), F32)
    zc = jnp.zeros((N, 16), F32)
    bidr = batch_ids.astype(jnp.int32).reshape(N // BN, 1, BN)
    Wm2p = _permute_wm2(Wm2)

    eaT = edge_attr.T.astype(BF16)
    eaE = eaT[:, 0::2]
    eaO = eaT[:, 1::2]
    qs = [_q_layer(eaE, eaO, Wm1[i, 2 * H:, :].astype(BF16))
          for i in range(DEPTH)]
    cnt16 = _cnt_kernel(dstr2, zc)

    h, pa, pb = _embed_pre(x, We1, be1.reshape(1, H), We2, be2.reshape(1, H),
                           Wm1[0, :H, :], Wm1[0, H:2 * H, :],
                           bm1[0].reshape(1, H))
    for i in range(DEPTH):
        pa2 = pa.reshape(2 * N, H // 4)
        pb2 = pb.reshape(2 * N, H // 4)
        S = _sc_layers[i](pa2, pb2, qs[i], gsrc, gdst, dstr, zrow)
        upd_args = (S, cnt16, h, Wm2p[i], bm2[i].reshape(1, H),
                    Wu1[i, :H, :], Wu1[i, H:, :], bu1[i].reshape(1, H),
                    Wu2[i], bu2[i].reshape(1, H))
        if i + 1 < DEPTH:
            h, pa, pb = _update_pre(*upd_args, Wm1[i + 1, :H, :],
                                    Wm1[i + 1, H:2 * H, :],
                                    bm1[i + 1].reshape(1, H))
        else:
            out = _update_pool(*upd_args, bidr, Wh1, bh1.reshape(1, H),
                               Wh2, bh2.reshape(1, OUT))
    return out


# speedup vs baseline: 2.2800x; 1.0888x over previous
import functools

import numpy as np

import jax
import jax.numpy as jnp
from jax import lax
from jax.experimental import pallas as pl
from jax.experimental.pallas import tpu as pltpu
from jax.experimental.pallas import tpu_sc as plsc

N = 10000
E = 160000
D = 256
DE = 16
H = 256
OUT = 128
DEPTH = 3
G = 64

NC = 2
NS = 16
EPT = E // NS
ROWS_PT = N // NS
K = 50
SK = 1000
CPS = SK // K
NSUP = EPT // SK
ER = E // K

BN = 2000
BE = 3200
F32 = jnp.float32
BF16 = jnp.bfloat16


def _permute_wm2(Wm2):
    w = Wm2.reshape(DEPTH, 2, 2, 4, 16, H)
    return jnp.transpose(w, (0, 1, 3, 2, 4, 5)).reshape(DEPTH, H, H)


def _pack_bf16_pairs(y):
    lo = jax.lax.bitcast_convert_type(
        y[:, :H // 4].astype(BF16), jnp.int16).astype(jnp.int32) & 0xFFFF
    hi = jax.lax.bitcast_convert_type(
        y[:, H // 4:].astype(BF16), jnp.int16).astype(jnp.int32) << 16
    return jax.lax.bitcast_convert_type(lo | hi, F32)


def _pack_pre(h, wa_ref, wb_ref, bm_ref, pa_ref, pb_ref):
    for c in range(2):
        wc = pl.ds(c * (H // 2), H // 2)
        pa_ref[c] = _pack_bf16_pairs(
            jnp.dot(h, wa_ref[:, wc], preferred_element_type=F32))
        pb_ref[c] = _pack_bf16_pairs(
            jnp.dot(h, wb_ref[:, wc], preferred_element_type=F32)
            + bm_ref[0, wc])


def _embed_pre_body(x_ref, w1_ref, b1_ref, w2_ref, b2_ref, wa_ref, wb_ref,
                    bm_ref, h_ref, pa_ref, pb_ref):
    h = jnp.maximum(
        jnp.dot(x_ref[...], w1_ref[...], preferred_element_type=F32) + b1_ref[0],
        0.0)
    h = jnp.dot(h, w2_ref[...], preferred_element_type=F32) + b2_ref[0]
    h_ref[...] = h
    _pack_pre(h, wa_ref, wb_ref, bm_ref, pa_ref, pb_ref)


_W_SPEC = pl.BlockSpec((H, H), lambda i: (0, 0))
_B_SPEC = pl.BlockSpec((1, H), lambda i: (0, 0))
_PK_SPEC = pl.BlockSpec((2, BN, H // 4), lambda i: (0, i, 0))
_PK_SHAPE = jax.ShapeDtypeStruct((2, N, H // 4), F32)


def _embed_pre(x, W1, b1, W2, b2, Wa, Wb, bm):
    return pl.pallas_call(
        _embed_pre_body,
        grid=(N // BN,),
        in_specs=[pl.BlockSpec((BN, D), lambda i: (i, 0)),
                  _W_SPEC, _B_SPEC, _W_SPEC, _B_SPEC,
                  _W_SPEC, _W_SPEC, _B_SPEC],
        out_specs=[pl.BlockSpec((BN, H), lambda i: (i, 0)),
                   _PK_SPEC, _PK_SPEC],
        out_shape=[jax.ShapeDtypeStruct((N, H), F32), _PK_SHAPE, _PK_SHAPE],
    )(x, W1, b1, W2, b2, Wa, Wb, bm)


def _q_body(eae_ref, eao_ref, wc_ref, q_ref):
    qe = jax.lax.dot_general(eae_ref[...], wc_ref[...],
                             (((0,), (0,)), ((), ())),
                             preferred_element_type=F32)
    qo = jax.lax.dot_general(eao_ref[...], wc_ref[...],
                             (((0,), (0,)), ((), ())),
                             preferred_element_type=F32)
    q_ref[...] = jnp.concatenate(
        [_pack_bf16_pairs(qe), _pack_bf16_pairs(qo)], axis=1)[None]


def _q_layer(eaE, eaO, wc):
    return pl.pallas_call(
        _q_body,
        grid=(2, E // 2 // BE),
        in_specs=[
            pl.BlockSpec((DE, BE), lambda c, e: (0, e)),
            pl.BlockSpec((DE, BE), lambda c, e: (0, e)),
            pl.BlockSpec((DE, H // 2), lambda c, e: (0, c)),
        ],
        out_specs=pl.BlockSpec((1, BE, H // 2), lambda c, e: (c, e, 0)),
        out_shape=jax.ShapeDtypeStruct((2, E // 2, H // 2), F32),
    )(eaE, eaO, wc)


def _upd_core(s_ref, cnt_ref, x_ref, wm2_ref, bm2_ref, wua_ref, wub_ref,
              bu1_ref, wu2_ref, bu2_ref):
    s0 = s_ref[0]
    s1 = s_ref[1]
    ssum = (jnp.dot(s0, wm2_ref[0:128, :], preferred_element_type=F32)
            + jnp.dot(s1, wm2_ref[128:256, :], preferred_element_type=F32))
    cnt = (cnt_ref[0] + cnt_ref[1])[:, 0:1]
    aggr = (ssum + cnt * bm2_ref[0]) / jnp.maximum(cnt, 1.0)
    xb = x_ref[...]
    h = jnp.maximum(
        jnp.dot(xb, wua_ref[...], preferred_element_type=F32)
        + jnp.dot(aggr, wub_ref[...], preferred_element_type=F32)
        + bu1_ref[0], 0.0)
    return jnp.dot(h, wu2_ref[...], preferred_element_type=F32) + bu2_ref[0]


def _upd_pre_body(s_ref, cnt_ref, x_ref, wm2_ref, bm2_ref, wua_ref, wub_ref,
                  bu1_ref, wu2_ref, bu2_ref, wa_ref, wb_ref, bm_ref,
                  h_ref, pa_ref, pb_ref):
    xn = _upd_core(s_ref, cnt_ref, x_ref, wm2_ref, bm2_ref, wua_ref, wub_ref,
                   bu1_ref, wu2_ref, bu2_ref)
    h_ref[...] = xn
    _pack_pre(xn, wa_ref, wb_ref, bm_ref, pa_ref, pb_ref)


_UPD_IN_SPECS = [
    pl.BlockSpec((2, BN, H // 2), lambda i: (0, i, 0)),
    pl.BlockSpec((NC, BN, 16), lambda i: (0, i, 0)),
    pl.BlockSpec((BN, H), lambda i: (i, 0)),
    _W_SPEC, _B_SPEC, _W_SPEC, _W_SPEC, _B_SPEC, _W_SPEC, _B_SPEC,
]


def _update_pre(S, cnt16, x, Wm2i, bm2i, Wua, Wub, bu1i, Wu2i, bu2i,
                Wa, Wb, bm):
    return pl.pallas_call(
        _upd_pre_body,
        grid=(N // BN,),
        in_specs=_UPD_IN_SPECS + [_W_SPEC, _W_SPEC, _B_SPEC],
        out_specs=[pl.BlockSpec((BN, H), lambda i: (i, 0)),
                   _PK_SPEC, _PK_SPEC],
        out_shape=[jax.ShapeDtypeStruct((N, H), F32), _PK_SHAPE, _PK_SHAPE],
    )(S, cnt16, x, Wm2i, bm2i, Wua, Wub, bu1i, Wu2i, bu2i, Wa, Wb, bm)


def _upd_pool_body(s_ref, cnt_ref, x_ref, wm2_ref, bm2_ref, wua_ref, wub_ref,
                   bu1_ref, wu2_ref, bu2_ref, bid_ref, wh1_ref, bh1_ref,
                   wh2_ref, bh2_ref, o_ref, acc_ref):
    i = pl.program_id(0)

    @pl.when(i == 0)
    def _init():
        acc_ref[...] = jnp.zeros_like(acc_ref)

    xn = _upd_core(s_ref, cnt_ref, x_ref, wm2_ref, bm2_ref, wua_ref, wub_ref,
                   bu1_ref, wu2_ref, bu2_ref)
    bid = bid_ref[0, 0]
    oh = (lax.broadcasted_iota(jnp.int32, (G, BN), 0)
          == bid[None, :]).astype(F32)
    acc_ref[...] += jnp.dot(oh, xn, preferred_element_type=F32)

    @pl.when(i == pl.num_programs(0) - 1)
    def _fin():
        h = jnp.maximum(
            jnp.dot(acc_ref[...], wh1_ref[...], preferred_element_type=F32)
            + bh1_ref[0], 0.0)
        o_ref[...] = jnp.dot(h, wh2_ref[...], preferred_element_type=F32) + bh2_ref[0]


def _update_pool(S, cnt16, x, Wm2i, bm2i, Wua, Wub, bu1i, Wu2i, bu2i,
                 bidr, Wh1, bh1, Wh2, bh2):
    return pl.pallas_call(
        _upd_pool_body,
        grid=(N // BN,),
        in_specs=_UPD_IN_SPECS + [
            pl.BlockSpec((1, 1, BN), lambda i: (i, 0, 0)),
            _W_SPEC, _B_SPEC,
            pl.BlockSpec((H, OUT), lambda i: (0, 0)),
            pl.BlockSpec((1, OUT), lambda i: (0, 0)),
        ],
        out_specs=pl.BlockSpec((G, OUT), lambda i: (0, 0)),
        out_shape=jax.ShapeDtypeStruct((G, OUT), F32),
        scratch_shapes=[pltpu.VMEM((G, H), F32)],
    )(S, cnt16, x, Wm2i, bm2i, Wua, Wub, bu1i, Wu2i, bu2i,
      bidr, Wh1, bh1, Wh2, bh2)


_MESH = plsc.VectorSubcoreMesh(core_axis_name="c", subcore_axis_name="s",
                               num_cores=NC, num_subcores=NS)
_SC_PARAMS = pltpu.CompilerParams(use_tc_tiling_on_sc=False,
                                  needs_layout_passes=False)


_KC = 100
_CROWS = E // _KC // (NC * NS)


def _cnt_body(dstr2_hbm, zc_hbm, cnt_out, sdst, vones, c_sh):
    cid = lax.axis_index("c")
    sid = lax.axis_index("s")
    myrows = pl.ds(sid * ROWS_PT, ROWS_PT)

    pltpu.sync_copy(zc_hbm.at[myrows], c_sh.at[myrows])

    def _ones_row(r, carry):
        vones[r] = jnp.ones((16,), F32)
        return carry
    lax.fori_loop(0, _KC, _ones_row, 0)
    plsc.subcore_barrier()

    base = (cid * NS + sid) * _CROWS
    pltpu.sync_copy(dstr2_hbm.at[pl.ds(base, _CROWS)], sdst)

    def chunk(c, carry):
        pltpu.sync_copy(vones, c_sh.at[sdst.at[c]], add=True)
        return carry
    lax.fori_loop(0, _CROWS, chunk, 0)
    plsc.subcore_barrier()

    pltpu.sync_copy(c_sh.at[myrows], cnt_out.at[cid, myrows])


_cnt_kernel = pl.kernel(
    _cnt_body,
    out_type=jax.ShapeDtypeStruct((NC, N, 16), F32),
    mesh=_MESH,
    scratch_types=[
        pltpu.VMEM((_CROWS, _KC), jnp.int32),
        pltpu.VMEM((_KC, 16), F32),
        pltpu.VMEM_SHARED((N, 16), F32),
    ],
    compiler_params=_SC_PARAMS)


def _make_sc(layer):
    scratch = [
        pltpu.VMEM((CPS, K), jnp.int32),
        pltpu.VMEM((CPS, K), jnp.int32),
        pltpu.VMEM((CPS, K), jnp.int32),
        pltpu.VMEM((2, K, H // 4), F32),
        pltpu.VMEM((2, K, H // 4), F32),
        pltpu.VMEM((2, K // 2, H // 2), F32),
        pltpu.VMEM((2, K, H // 2), F32),
        pltpu.VMEM_SHARED((N, H // 2), F32),
        pltpu.SemaphoreType.DMA,
        pltpu.SemaphoreType.DMA,
        pltpu.SemaphoreType.DMA,
    ]

    def body(pa_hbm, pb_hbm, ql_hbm, gsrc_hbm, gdst_hbm, dstr_hbm, z_hbm,
             s_out, isrc, idst, sdst, va, vb, vq, stg, s_sh, sem1, sem2,
             sem3):
        cid = lax.axis_index("c")
        sid = lax.axis_index("s")
        myrows = pl.ds(sid * ROWS_PT, ROWS_PT)

        pltpu.sync_copy(z_hbm.at[myrows], s_sh.at[myrows])
        plsc.subcore_barrier()

        def fire(sup_base_e, b):
            buf = b % 2
            cps = [
                pltpu.async_copy(pa_hbm.at[isrc.at[b]], va.at[buf], sem1),
                pltpu.async_copy(pb_hbm.at[idst.at[b]], vb.at[buf], sem1),
                pltpu.async_copy(
                    ql_hbm.at[cid, pl.ds((sup_base_e + b * K) // 2, K // 2)],
                    vq.at[buf], sem2),
            ]
            return cps

        def crunch(b):
            buf = b % 2

            @plsc.parallel_loop(0, K // 2, unroll=5)
            def rowf(r2):
                for p in range(2):
                    r = 2 * r2 + p
                    for g in range(H // 2 // 32):
                        sl = pl.ds(g * 16, 16)
                        a32 = plsc.bitcast(va[buf, r, sl], BF16)
                        b32 = plsc.bitcast(vb[buf, r, sl], BF16)
                        q32 = plsc.bitcast(
                            vq[buf, r2, pl.ds(p * 64 + g * 16, 16)], BF16)
                        v = jnp.maximum(a32 + b32 + q32,
                                        jnp.zeros((32,), BF16))
                        lo, hi = plsc.unpack(
                            v, format=plsc.PackFormat.INTERLEAVED)
                        stg[buf, r, pl.ds(g * 32, 16)] = lo
                        stg[buf, r, pl.ds(g * 32 + 16, 16)] = hi

        def super_loop(s, carry):
            base_row = sid * (EPT // K) + s * CPS
            base_e = sid * EPT + s * SK
            pltpu.sync_copy(gsrc_hbm.at[cid, pl.ds(base_row, CPS)], isrc)
            pltpu.sync_copy(gdst_hbm.at[cid, pl.ds(base_row, CPS)], idst)
            pltpu.sync_copy(dstr_hbm.at[pl.ds(base_row, CPS)], sdst)
            cps = fire(base_e, 0)
            scats = [None, None]
            for b in range(CPS):
                for cp in cps:
                    cp.wait()
                if b + 1 < CPS:
                    cps = fire(base_e, b + 1)
                if scats[b % 2] is not None:
                    scats[b % 2].wait()
                crunch(b)
                scats[b % 2] = pltpu.async_copy(
                    stg.at[b % 2], s_sh.at[sdst.at[b]], sem3, add=True)
            scats[0].wait()
            scats[1].wait()
            return carry
        lax.fori_loop(0, NSUP, super_loop, 0)
        plsc.subcore_barrier()

        pltpu.sync_copy(s_sh.at[myrows], s_out.at[cid, myrows])

    return pl.kernel(body,
                     out_type=jax.ShapeDtypeStruct((2, N, H // 2), F32),
                     mesh=_MESH, scratch_types=scratch,
                     compiler_params=_SC_PARAMS)


_sc_layers = [_make_sc(i) for i in range(DEPTH)]


def kernel(x, edge_index, edge_attr, batch_ids, We1, be1, We2, be2,
           Wm1, bm1, Wm2, bm2, Wu1, bu1, Wu2, bu2, Wh1, bh1, Wh2, bh2):
    src = edge_index[0].astype(jnp.int32)
    dst = edge_index[1].astype(jnp.int32)
    gsrc = jnp.stack([src, N + src]).reshape(2, ER, K)
    gdst = jnp.stack([dst, N + dst]).reshape(2, ER, K)
    dstr = dst.reshape(ER, K)
    dstr2 = dst.reshape(E // _KC, _KC)
    zrow = jnp.zeros((N, H // 2), F32)
    zc = jnp.zeros((N, 16), F32)
    bidr = batch_ids.astype(jnp.int32).reshape(N // BN, 1, BN)
    Wm2p = _permute_wm2(Wm2)

    eaT = edge_attr.T.astype(BF16)
    eaE = eaT[:, 0::2]
    eaO = eaT[:, 1::2]
    qs = [_q_layer(eaE, eaO, Wm1[i, 2 * H:, :].astype(BF16))
          for i in range(DEPTH)]
    cnt16 = _cnt_kernel(dstr2, zc)

    h, pa, pb = _embed_pre(x, We1, be1.reshape(1, H), We2, be2.reshape(1, H),
                           Wm1[0, :H, :], Wm1[0, H:2 * H, :],
                           bm1[0].reshape(1, H))
    for i in range(DEPTH):
        pa2 = pa.reshape(2 * N, H // 4)
        pb2 = pb.reshape(2 * N, H // 4)
        S = _sc_layers[i](pa2, pb2, qs[i], gsrc, gdst, dstr, zrow)
        upd_args = (S, cnt16, h, Wm2p[i], bm2[i].reshape(1, H),
                    Wu1[i, :H, :], Wu1[i, H:, :], bu1[i].reshape(1, H),
                    Wu2[i], bu2[i].reshape(1, H))
        if i + 1 < DEPTH:
            h, pa, pb = _update_pre(*upd_args, Wm1[i + 1, :H, :],
                                    Wm1[i + 1, H:2 * H, :],
                                    bm1[i + 1].reshape(1, H))
        else:
            out = _update_pool(*upd_args, bidr, Wh1, bh1.reshape(1, H),
                               Wh2, bh2.reshape(1, OUT))
    return out
